# Initial kernel scaffold; baseline (speedup 1.0000x reference)
import jax
import jax.numpy as jnp
from jax.experimental import pallas as pl


def kernel(x_agv, x_picker, x_location, src0, dst0, src1, dst1, src2, dst2, src3, dst3, src4, dst4, src5, dst5, params):
    raise NotImplementedError("write your pallas kernel here")


# speedup vs baseline: 1.7842x; 1.7842x over previous
import functools

import jax
import jax.numpy as jnp
from jax import lax
from jax.experimental import pallas as pl
from jax.experimental.pallas import tpu as pltpu
from jax.experimental.pallas import tpu_sc as plsc

H = 128
OH = 64
A = 16
N_AGV = 20000
N_PICKER = 5000
N_LOC = 20000
E = 320000
NNODE = {"agv": N_AGV, "picker": N_PICKER, "location": N_LOC}
RELS = [("agv", "location"), ("location", "agv"), ("agv", "agv"),
        ("picker", "location"), ("agv", "picker"), ("picker", "agv")]
SRC_OF = {"agv": [0, 2, 4], "location": [1], "picker": [3, 5]}
DST_OF = {"location": [0, 3], "agv": [1, 2, 5], "picker": [4]}
TYPES = ["agv", "picker", "location"]

NC, NS = 2, 16
EP = 327680
EROWS = EP // 128
ROWS_PER_TILE = EROWS // (NC * NS)
G = 4
NGROUP = ROWS_PER_TILE // G
ACC_ROWS = 20008
DUMMY = 20000
RPT = 1256

_QSIZES = []
for _r in range(6):
    for _h in range(2):
        _QSIZES.append(NNODE[RELS[_r][0]])
TOFF = [0]
for _s in _QSIZES:
    TOFF.append(TOFF[-1] + _s)

DOFF = [0, 20000, 40000, 60000, 80000, 85000]
DTOT = 105000


def _rel_params(r):
    nd = jnp.where(r == 4, 5000, 20000)
    roff = 20000 * jnp.minimum(r, 4) + 5000 * jnp.maximum(r - 4, 0)
    return nd, roff


def _segsum_body(tbl, srcq, dstr, zf, s_out, acc, idx_s, idx_d, rows, sem):
    c = lax.axis_index("c")
    s = lax.axis_index("s")
    row0 = (c * NS + s) * ROWS_PER_TILE

    def q_body(q, carry):
        r = q // 2
        h = q % 2
        nd, roff = _rel_params(r)
        zstart = jnp.minimum(s * RPT, (nd + 8) - RPT)
        pltpu.sync_copy(zf.at[pl.ds(zstart, RPT)], acc.at[pl.ds(zstart, RPT)])
        plsc.subcore_barrier()

        def g_body(g, carry2):
            roff_rows = row0 + g * G
            pltpu.sync_copy(srcq.at[q].at[pl.ds(roff_rows, G)], idx_s)
            pltpu.sync_copy(dstr.at[r].at[pl.ds(roff_rows, G)], idx_d)
            cps = [
                pltpu.async_copy(tbl.at[idx_s.at[j]],
                                 rows.at[pl.ds(j * 128, 128)], sem)
                for j in range(G)
            ]
            for cp in cps:
                cp.wait()
            for j in range(G):
                pltpu.sync_copy(rows.at[pl.ds(j * 128, 128)],
                                acc.at[idx_d.at[j]], add=True)
            return carry2

        lax.fori_loop(0, NGROUP, g_body, 0)
        plsc.subcore_barrier()
        dstart = jnp.minimum(s * RPT, nd - RPT)
        pltpu.sync_copy(acc.at[pl.ds(dstart, RPT)],
                        s_out.at[c].at[h].at[pl.ds(roff + dstart, RPT)])
        plsc.subcore_barrier()
        return carry

    lax.fori_loop(0, 12, q_body, 0)


def _counts_body(dstr, zc, ones_h, c_out, acc, idx_d, ones_v, sem):
    del sem
    c = lax.axis_index("c")
    s = lax.axis_index("s")
    row0 = (c * NS + s) * ROWS_PER_TILE
    pltpu.sync_copy(ones_h, ones_v)

    def r_body(r, carry):
        nd, roff = _rel_params(r)
        zstart = jnp.minimum(s * RPT, (nd + 8) - RPT)
        pltpu.sync_copy(zc.at[pl.ds(zstart, RPT)], acc.at[pl.ds(zstart, RPT)])
        plsc.subcore_barrier()

        def g_body(g, carry2):
            roff_rows = row0 + g * 8
            pltpu.sync_copy(dstr.at[r].at[pl.ds(roff_rows, 8)], idx_d)
            for j in range(8):
                pltpu.sync_copy(ones_v, acc.at[idx_d.at[j]], add=True)
            return carry2

        lax.fori_loop(0, ROWS_PER_TILE // 8, g_body, 0)
        plsc.subcore_barrier()
        dstart = jnp.minimum(s * RPT, nd - RPT)
        pltpu.sync_copy(acc.at[pl.ds(dstart, RPT)],
                        c_out.at[c].at[pl.ds(roff + dstart, RPT)])
        plsc.subcore_barrier()
        return carry

    lax.fori_loop(0, 6, r_body, 0)


_SC_MESH = plsc.VectorSubcoreMesh(core_axis_name="c", subcore_axis_name="s")
_SC_PARAMS = pltpu.CompilerParams(use_tc_tiling_on_sc=False)

_segsum_sc = pl.kernel(
    _segsum_body,
    out_type=jax.ShapeDtypeStruct((NC, 2, DTOT, 64), jnp.float32),
    mesh=_SC_MESH,
    scratch_types=[
        pltpu.VMEM_SHARED((ACC_ROWS, 64), jnp.float32),
        pltpu.VMEM((G, 128), jnp.int32),
        pltpu.VMEM((G, 128), jnp.int32),
        pltpu.VMEM((G * 128, 64), jnp.float32),
        pltpu.SemaphoreType.DMA,
    ],
    compiler_params=_SC_PARAMS,
)

_counts_sc = pl.kernel(
    _counts_body,
    out_type=jax.ShapeDtypeStruct((NC, DTOT, 16), jnp.float32),
    mesh=_SC_MESH,
    scratch_types=[
        pltpu.VMEM_SHARED((ACC_ROWS, 16), jnp.float32),
        pltpu.VMEM((8, 128), jnp.int32),
        pltpu.VMEM((128, 16), jnp.float32),
        pltpu.SemaphoreType.DMA,
    ],
    compiler_params=_SC_PARAMS,
)


def _mm_stack(x, wstack, bstack, bm=1000):
    n, kin = x.shape
    S = wstack.shape[0]

    def body(x_ref, w_ref, b_ref, o_ref):
        o_ref[0] = (jnp.dot(x_ref[...], w_ref[0],
                            preferred_element_type=jnp.float32) + b_ref[0])

    return pl.pallas_call(
        body,
        grid=(S, n // bm),
        in_specs=[
            pl.BlockSpec((bm, kin), lambda j, i: (i, 0)),
            pl.BlockSpec((1, kin, 64), lambda j, i: (j, 0, 0)),
            pl.BlockSpec((1, 1, 64), lambda j, i: (j, 0, 0)),
        ],
        out_specs=pl.BlockSpec((1, bm, 64), lambda j, i: (j, i, 0)),
        out_shape=jax.ShapeDtypeStruct((S, n, 64), jnp.float32),
    )(x, wstack, bstack.reshape(S, 1, 64))


def _combine(s_all, c_all, y_self, bsum, rel_offs, kd, n, bm=1000):
    nr = len(rel_offs)
    self_blk = (y_self.shape[0] - 2) // 2

    def body(*refs):
        s_refs = refs[:nr]
        c_refs = refs[nr:2 * nr]
        sref, bref, oref = refs[2 * nr], refs[2 * nr + 1], refs[2 * nr + 2]
        tot = jnp.concatenate([sref[0], sref[1]], axis=-1) + bref[...]
        for s_ref, c_ref in zip(s_refs, c_refs):
            m = jnp.concatenate([s_ref[0, 0] + s_ref[1, 0],
                                 s_ref[0, 1] + s_ref[1, 1]], axis=-1)
            cc = c_ref[0, :, 0] + c_ref[1, :, 0]
            tot = tot + m * (1.0 / jnp.maximum(cc, 1.0))[:, None]
        oref[...] = jnp.maximum(tot * (1.0 / kd), 0.0)

    in_specs = []
    for off in rel_offs:
        blk = off // bm
        in_specs.append(pl.BlockSpec((NC, 2, bm, 64),
                                     lambda i, blk=blk: (0, 0, blk + i, 0)))
    for off in rel_offs:
        blk = off // bm
        in_specs.append(pl.BlockSpec((NC, bm, 16),
                                     lambda i, blk=blk: (0, blk + i, 0)))
    in_specs.append(pl.BlockSpec((2, bm, 64), lambda i: (self_blk, i, 0)))
    in_specs.append(pl.BlockSpec((1, H), lambda i: (0, 0)))

    return pl.pallas_call(
        body,
        grid=(n // bm,),
        in_specs=in_specs,
        out_specs=pl.BlockSpec((bm, H), lambda i: (i, 0)),
        out_shape=jax.ShapeDtypeStruct((n, H), jnp.float32),
    )(*([s_all] * nr + [c_all] * nr + [y_self, bsum]))


def _ode(z, w1, b1, w2, b2, w3, b3, bm=1000):
    n = z.shape[0]

    def body(z_ref, w1r, b1r, w2r, b2r, w3r, b3r, o_ref):
        def f(h):
            h1 = jnp.tanh(jnp.dot(h, w1r[...],
                                  preferred_element_type=jnp.float32) + b1r[...])
            h2 = jnp.tanh(jnp.dot(h1, w2r[...],
                                  preferred_element_type=jnp.float32) + b2r[...])
            return jnp.dot(h2, w3r[...],
                           preferred_element_type=jnp.float32) + b3r[...]

        dt = 0.1

        def step(i, zz):
            k1 = f(zz)
            k2 = f(zz + (0.5 * dt) * k1)
            k3 = f(zz + (0.5 * dt) * k2)
            k4 = f(zz + dt * k3)
            return zz + (dt / 6.0) * (k1 + 2.0 * k2 + 2.0 * k3 + k4)

        o_ref[...] = lax.fori_loop(0, 10, step, z_ref[...])

    full = lambda shape: pl.BlockSpec(shape, lambda i: tuple(0 for _ in shape))
    return pl.pallas_call(
        body,
        grid=(n // bm,),
        in_specs=[
            pl.BlockSpec((bm, H), lambda i: (i, 0)),
            full((H, OH)), full((1, OH)), full((OH, OH)), full((1, OH)),
            full((OH, H)), full((1, H)),
        ],
        out_specs=pl.BlockSpec((bm, H), lambda i: (i, 0)),
        out_shape=jax.ShapeDtypeStruct((n, H), jnp.float32),
    )(z, w1, b1, w2, b2, w3, b3)


def _head(x, w1, b1, w2, b2, bm=1000):
    n = x.shape[0]

    def body(x_ref, w1r, b1r, w2r, b2r, o_ref):
        h = jnp.maximum(jnp.dot(x_ref[...], w1r[...],
                                preferred_element_type=jnp.float32) + b1r[...],
                        0.0)
        o_ref[...] = jnp.dot(h, w2r[...],
                             preferred_element_type=jnp.float32) + b2r[...]

    full = lambda shape: pl.BlockSpec(shape, lambda i: tuple(0 for _ in shape))
    return pl.pallas_call(
        body,
        grid=(n // bm,),
        in_specs=[
            pl.BlockSpec((bm, H), lambda i: (i, 0)),
            full((H, OH)), full((1, OH)), full((OH, A)), full((1, A)),
        ],
        out_specs=pl.BlockSpec((bm, A), lambda i: (i, 0)),
        out_shape=jax.ShapeDtypeStruct((n, A), jnp.float32),
    )(x, w1, b1, w2, b2)


def kernel(x_agv, x_picker, x_location, src0, dst0, src1, dst1, src2, dst2,
           src3, dst3, src4, dst4, src5, dst5, params):
    p = params
    convs = p["convs"]
    srcs = [src0, src1, src2, src3, src4, src5]
    dsts = [dst0, dst1, dst2, dst3, dst4, dst5]
    x_raw = {"agv": x_agv, "picker": x_picker, "location": x_location}

    pad_s = jnp.zeros((EP - E,), jnp.int32)
    pad_d = jnp.full((EP - E,), DUMMY, jnp.int32)
    src_p = [jnp.concatenate([s, pad_s]) for s in srcs]
    dst_p = [jnp.concatenate([d, pad_d]) for d in dsts]
    srcq = jnp.stack([
        (src_p[q // 2] + TOFF[q]).reshape(EROWS, 128) for q in range(12)
    ])
    dstr = jnp.stack([d.reshape(EROWS, 128) for d in dst_p])

    zf = jnp.zeros((ACC_ROWS, 64), jnp.float32)
    zc = jnp.zeros((ACC_ROWS, 16), jnp.float32)
    ones_h = jnp.ones((128, 16), jnp.float32)

    c_all = _counts_sc(dstr, zc, ones_h)

    def stacks(l):
        wst, bst = {}, {}
        for t in TYPES:
            mats, biases = [], []
            rels_s = SRC_OF[t]
            wr_sum = sum(convs[l][r]["Wr"] for r in DST_OF[t])
            if l == 0:
                din = x_raw[t].shape[1]
                we = jnp.pad(p["emb_" + t]["W"], ((0, 8 - din), (0, 0)))
                be = p["emb_" + t]["b"]
                srcmats = [we @ convs[l][r]["Wl"] for r in rels_s]
                srcbias = [be @ convs[l][r]["Wl"] for r in rels_s]
                selfmat, selfbias = we @ wr_sum, be @ wr_sum
            else:
                srcmats = [convs[l][r]["Wl"] for r in rels_s]
                srcbias = [jnp.zeros((H,), jnp.float32) for r in rels_s]
                selfmat = wr_sum
                selfbias = jnp.zeros((H,), jnp.float32)
            for m, b in zip(srcmats + [selfmat], srcbias + [selfbias]):
                mats += [m[:, :64], m[:, 64:]]
                biases += [b[:64], b[64:]]
            wst[t] = jnp.stack(mats)
            bst[t] = jnp.stack(biases)
        return wst, bst

    x = {t: jnp.pad(x_raw[t], ((0, 0), (0, 8 - x_raw[t].shape[1])))
         for t in TYPES}
    for l in range(2):
        wst, bst = stacks(l)
        y = {t: _mm_stack(x[t], wst[t], bst[t]) for t in TYPES}
        tables = []
        for q in range(12):
            r = q // 2
            t = RELS[r][0]
            pos = SRC_OF[t].index(r)
            tables.append(y[t][2 * pos + q % 2])
        tbl = jnp.concatenate(tables, axis=0)
        s_all = _segsum_sc(tbl, srcq, dstr, zf)
        xn = {}
        for t in TYPES:
            bsum = sum(convs[l][r]["bl"] for r in DST_OF[t]).reshape(1, H)
            xn[t] = _combine(s_all, c_all, y[t], bsum,
                             [DOFF[r] for r in DST_OF[t]],
                             float(len(DST_OF[t])), NNODE[t])
        x = xn

    z = jnp.concatenate([x["agv"], x["picker"], x["location"]], axis=0)
    z = _ode(z,
             p["ode1"]["W"], p["ode1"]["b"].reshape(1, OH),
             p["ode2"]["W"], p["ode2"]["b"].reshape(1, OH),
             p["ode3"]["W"], p["ode3"]["b"].reshape(1, H))
    agv_e = z[:N_AGV]
    picker_e = z[N_AGV:N_AGV + N_PICKER]
    loc_e = z[N_AGV + N_PICKER:]
    agv_q = _head(agv_e, p["agv_h1"]["W"], p["agv_h1"]["b"].reshape(1, OH),
                  p["agv_h2"]["W"], p["agv_h2"]["b"].reshape(1, A))
    picker_q = _head(picker_e, p["picker_h1"]["W"],
                     p["picker_h1"]["b"].reshape(1, OH),
                     p["picker_h2"]["W"], p["picker_h2"]["b"].reshape(1, A))
    return (agv_q, picker_q, agv_e, picker_e, loc_e)


# speedup vs baseline: 1.8791x; 1.0532x over previous
import functools

import jax
import jax.numpy as jnp
from jax import lax
from jax.experimental import pallas as pl
from jax.experimental.pallas import tpu as pltpu
from jax.experimental.pallas import tpu_sc as plsc

H = 128
OH = 64
A = 16
N_AGV = 20000
N_PICKER = 5000
N_LOC = 20000
E = 320000
NNODE = {"agv": N_AGV, "picker": N_PICKER, "location": N_LOC}
RELS = [("agv", "location"), ("location", "agv"), ("agv", "agv"),
        ("picker", "location"), ("agv", "picker"), ("picker", "agv")]
SRC_OF = {"agv": [0, 2, 4], "location": [1], "picker": [3, 5]}
DST_OF = {"location": [0, 3], "agv": [1, 2, 5], "picker": [4]}
TYPES = ["agv", "picker", "location"]

NC, NS = 2, 16
EP = 327680
EROWS = EP // 128
ROWS_PER_TILE = EROWS // (NC * NS)
G = 2
NGROUP = ROWS_PER_TILE // G
ACC_ROWS = 20008
DUMMY = 20000
RPT = 1256

_QSIZES = []
for _r in range(6):
    for _h in range(2):
        _QSIZES.append(NNODE[RELS[_r][0]])
TOFF = [0]
for _s in _QSIZES:
    TOFF.append(TOFF[-1] + _s)

DOFF = [0, 20000, 40000, 60000, 80000, 85000]
DTOT = 105000


def _rel_params(r):
    nd = jnp.where(r == 4, 5000, 20000)
    roff = 20000 * jnp.minimum(r, 4) + 5000 * jnp.maximum(r - 4, 0)
    return nd, roff


def _segsum_body(tbl, idxc, zf, s_out, acc, idxv, rows, sem_g, ss0, ss1):
    c = lax.axis_index("c")
    s = lax.axis_index("s")
    tile = c * NS + s
    gid0 = tile * NGROUP

    def do_group(q, gg, b, ssem):
        pltpu.sync_copy(idxc.at[q].at[gg], idxv.at[b])
        cps = [
            pltpu.async_copy(tbl.at[idxv.at[b].at[j]],
                             rows.at[b].at[pl.ds(j * 128, 128)], sem_g)
            for j in range(G)
        ]
        for cp in cps:
            cp.wait()
        for j in range(G):
            pltpu.async_copy(rows.at[b].at[pl.ds(j * 128, 128)],
                             acc.at[idxv.at[b].at[G + j]], ssem, add=True)

    def drain(b, ssem):
        for j in range(G):
            pltpu.make_async_copy(
                rows.at[b].at[pl.ds(j * 128, 128)],
                acc.at[idxv.at[b].at[G + j]], ssem).wait()

    def q_body(q, carry):
        r = q // 2
        h = q % 2
        nd, roff = _rel_params(r)
        zstart = jnp.minimum(s * RPT, (nd + 8) - RPT)
        pltpu.sync_copy(zf.at[pl.ds(zstart, RPT)], acc.at[pl.ds(zstart, RPT)])
        plsc.subcore_barrier()

        do_group(q, gid0 + 0, 0, ss0)
        do_group(q, gid0 + 1, 1, ss1)

        def g_body(g2, carry2):
            drain(0, ss0)
            do_group(q, gid0 + 2 * g2, 0, ss0)
            drain(1, ss1)
            do_group(q, gid0 + 2 * g2 + 1, 1, ss1)
            return carry2

        lax.fori_loop(1, NGROUP // 2, g_body, 0)
        drain(0, ss0)
        drain(1, ss1)
        plsc.subcore_barrier()
        dstart = jnp.minimum(s * RPT, nd - RPT)
        pltpu.sync_copy(acc.at[pl.ds(dstart, RPT)],
                        s_out.at[c].at[h].at[pl.ds(roff + dstart, RPT)])
        plsc.subcore_barrier()
        return carry

    lax.fori_loop(0, 12, q_body, 0)


def _counts_body(dstr, zc, ones_h, c_out, acc, idx_d, ones_v, sem):
    del sem
    c = lax.axis_index("c")
    s = lax.axis_index("s")
    row0 = (c * NS + s) * ROWS_PER_TILE
    pltpu.sync_copy(ones_h, ones_v)

    def r_body(r, carry):
        nd, roff = _rel_params(r)
        zstart = jnp.minimum(s * RPT, (nd + 8) - RPT)
        pltpu.sync_copy(zc.at[pl.ds(zstart, RPT)], acc.at[pl.ds(zstart, RPT)])
        plsc.subcore_barrier()

        def g_body(g, carry2):
            roff_rows = row0 + g * 8
            pltpu.sync_copy(dstr.at[r].at[pl.ds(roff_rows, 8)], idx_d)
            for j in range(8):
                pltpu.sync_copy(ones_v, acc.at[idx_d.at[j]], add=True)
            return carry2

        lax.fori_loop(0, ROWS_PER_TILE // 8, g_body, 0)
        plsc.subcore_barrier()
        dstart = jnp.minimum(s * RPT, nd - RPT)
        pltpu.sync_copy(acc.at[pl.ds(dstart, RPT)],
                        c_out.at[c].at[pl.ds(roff + dstart, RPT)])
        plsc.subcore_barrier()
        return carry

    lax.fori_loop(0, 6, r_body, 0)


_SC_MESH = plsc.VectorSubcoreMesh(core_axis_name="c", subcore_axis_name="s")
_SC_PARAMS = pltpu.CompilerParams(use_tc_tiling_on_sc=False)

_segsum_sc = pl.kernel(
    _segsum_body,
    out_type=jax.ShapeDtypeStruct((NC, 2, DTOT, 64), jnp.float32),
    mesh=_SC_MESH,
    scratch_types=[
        pltpu.VMEM_SHARED((ACC_ROWS, 64), jnp.float32),
        pltpu.VMEM((2, 2 * G, 128), jnp.int32),
        pltpu.VMEM((2, G * 128, 64), jnp.float32),
        pltpu.SemaphoreType.DMA,
        pltpu.SemaphoreType.DMA,
        pltpu.SemaphoreType.DMA,
    ],
    compiler_params=_SC_PARAMS,
)

_counts_sc = pl.kernel(
    _counts_body,
    out_type=jax.ShapeDtypeStruct((NC, DTOT, 16), jnp.float32),
    mesh=_SC_MESH,
    scratch_types=[
        pltpu.VMEM_SHARED((ACC_ROWS, 16), jnp.float32),
        pltpu.VMEM((8, 128), jnp.int32),
        pltpu.VMEM((128, 16), jnp.float32),
        pltpu.SemaphoreType.DMA,
    ],
    compiler_params=_SC_PARAMS,
)


def _mm_stack(x, wstack, bstack, bm=1000):
    n, kin = x.shape
    S = wstack.shape[0]

    def body(x_ref, w_ref, b_ref, o_ref):
        o_ref[0] = (jnp.dot(x_ref[...], w_ref[0],
                            preferred_element_type=jnp.float32) + b_ref[0])

    return pl.pallas_call(
        body,
        grid=(S, n // bm),
        in_specs=[
            pl.BlockSpec((bm, kin), lambda j, i: (i, 0)),
            pl.BlockSpec((1, kin, 64), lambda j, i: (j, 0, 0)),
            pl.BlockSpec((1, 1, 64), lambda j, i: (j, 0, 0)),
        ],
        out_specs=pl.BlockSpec((1, bm, 64), lambda j, i: (j, i, 0)),
        out_shape=jax.ShapeDtypeStruct((S, n, 64), jnp.float32),
    )(x, wstack, bstack.reshape(S, 1, 64))


def _combine(s_all, c_all, y_self, bsum, rel_offs, kd, n, bm=1000):
    nr = len(rel_offs)
    self_blk = (y_self.shape[0] - 2) // 2

    def body(*refs):
        s_refs = refs[:nr]
        c_refs = refs[nr:2 * nr]
        sref, bref, oref = refs[2 * nr], refs[2 * nr + 1], refs[2 * nr + 2]
        tot = jnp.concatenate([sref[0], sref[1]], axis=-1) + bref[...]
        for s_ref, c_ref in zip(s_refs, c_refs):
            m = jnp.concatenate([s_ref[0, 0] + s_ref[1, 0],
                                 s_ref[0, 1] + s_ref[1, 1]], axis=-1)
            cc = c_ref[0, :, 0] + c_ref[1, :, 0]
            tot = tot + m * (1.0 / jnp.maximum(cc, 1.0))[:, None]
        oref[...] = jnp.maximum(tot * (1.0 / kd), 0.0)

    in_specs = []
    for off in rel_offs:
        blk = off // bm
        in_specs.append(pl.BlockSpec((NC, 2, bm, 64),
                                     lambda i, blk=blk: (0, 0, blk + i, 0)))
    for off in rel_offs:
        blk = off // bm
        in_specs.append(pl.BlockSpec((NC, bm, 16),
                                     lambda i, blk=blk: (0, blk + i, 0)))
    in_specs.append(pl.BlockSpec((2, bm, 64), lambda i: (self_blk, i, 0)))
    in_specs.append(pl.BlockSpec((1, H), lambda i: (0, 0)))

    return pl.pallas_call(
        body,
        grid=(n // bm,),
        in_specs=in_specs,
        out_specs=pl.BlockSpec((bm, H), lambda i: (i, 0)),
        out_shape=jax.ShapeDtypeStruct((n, H), jnp.float32),
    )(*([s_all] * nr + [c_all] * nr + [y_self, bsum]))


def _ode(z, w1, b1, w2, b2, w3, b3, bm=1000):
    n = z.shape[0]

    def body(z_ref, w1r, b1r, w2r, b2r, w3r, b3r, o_ref):
        def f(h):
            h1 = jnp.tanh(jnp.dot(h, w1r[...],
                                  preferred_element_type=jnp.float32) + b1r[...])
            h2 = jnp.tanh(jnp.dot(h1, w2r[...],
                                  preferred_element_type=jnp.float32) + b2r[...])
            return jnp.dot(h2, w3r[...],
                           preferred_element_type=jnp.float32) + b3r[...]

        dt = 0.1

        def step(i, zz):
            k1 = f(zz)
            k2 = f(zz + (0.5 * dt) * k1)
            k3 = f(zz + (0.5 * dt) * k2)
            k4 = f(zz + dt * k3)
            return zz + (dt / 6.0) * (k1 + 2.0 * k2 + 2.0 * k3 + k4)

        o_ref[...] = lax.fori_loop(0, 10, step, z_ref[...])

    full = lambda shape: pl.BlockSpec(shape, lambda i: tuple(0 for _ in shape))
    return pl.pallas_call(
        body,
        grid=(n // bm,),
        in_specs=[
            pl.BlockSpec((bm, H), lambda i: (i, 0)),
            full((H, OH)), full((1, OH)), full((OH, OH)), full((1, OH)),
            full((OH, H)), full((1, H)),
        ],
        out_specs=pl.BlockSpec((bm, H), lambda i: (i, 0)),
        out_shape=jax.ShapeDtypeStruct((n, H), jnp.float32),
    )(z, w1, b1, w2, b2, w3, b3)


def _head(x, w1, b1, w2, b2, bm=1000):
    n = x.shape[0]

    def body(x_ref, w1r, b1r, w2r, b2r, o_ref):
        h = jnp.maximum(jnp.dot(x_ref[...], w1r[...],
                                preferred_element_type=jnp.float32) + b1r[...],
                        0.0)
        o_ref[...] = jnp.dot(h, w2r[...],
                             preferred_element_type=jnp.float32) + b2r[...]

    full = lambda shape: pl.BlockSpec(shape, lambda i: tuple(0 for _ in shape))
    return pl.pallas_call(
        body,
        grid=(n // bm,),
        in_specs=[
            pl.BlockSpec((bm, H), lambda i: (i, 0)),
            full((H, OH)), full((1, OH)), full((OH, A)), full((1, A)),
        ],
        out_specs=pl.BlockSpec((bm, A), lambda i: (i, 0)),
        out_shape=jax.ShapeDtypeStruct((n, A), jnp.float32),
    )(x, w1, b1, w2, b2)


def kernel(x_agv, x_picker, x_location, src0, dst0, src1, dst1, src2, dst2,
           src3, dst3, src4, dst4, src5, dst5, params):
    p = params
    convs = p["convs"]
    srcs = [src0, src1, src2, src3, src4, src5]
    dsts = [dst0, dst1, dst2, dst3, dst4, dst5]
    x_raw = {"agv": x_agv, "picker": x_picker, "location": x_location}

    pad_s = jnp.zeros((EP - E,), jnp.int32)
    pad_d = jnp.full((EP - E,), DUMMY, jnp.int32)
    src_p = [jnp.concatenate([s, pad_s]) for s in srcs]
    dst_p = [jnp.concatenate([d, pad_d]) for d in dsts]
    dstr = jnp.stack([d.reshape(EROWS, 128) for d in dst_p])
    idxc = jnp.stack([
        jnp.concatenate([
            (src_p[q // 2] + TOFF[q]).reshape(EROWS // G, G, 128),
            dst_p[q // 2].reshape(EROWS // G, G, 128),
        ], axis=1) for q in range(12)
    ])

    zf = jnp.zeros((ACC_ROWS, 64), jnp.float32)
    zc = jnp.zeros((ACC_ROWS, 16), jnp.float32)
    ones_h = jnp.ones((128, 16), jnp.float32)

    c_all = _counts_sc(dstr, zc, ones_h)

    def stacks(l):
        wst, bst = {}, {}
        for t in TYPES:
            mats, biases = [], []
            rels_s = SRC_OF[t]
            wr_sum = sum(convs[l][r]["Wr"] for r in DST_OF[t])
            if l == 0:
                din = x_raw[t].shape[1]
                we = jnp.pad(p["emb_" + t]["W"], ((0, 8 - din), (0, 0)))
                be = p["emb_" + t]["b"]
                srcmats = [we @ convs[l][r]["Wl"] for r in rels_s]
                srcbias = [be @ convs[l][r]["Wl"] for r in rels_s]
                selfmat, selfbias = we @ wr_sum, be @ wr_sum
            else:
                srcmats = [convs[l][r]["Wl"] for r in rels_s]
                srcbias = [jnp.zeros((H,), jnp.float32) for r in rels_s]
                selfmat = wr_sum
                selfbias = jnp.zeros((H,), jnp.float32)
            for m, b in zip(srcmats + [selfmat], srcbias + [selfbias]):
                mats += [m[:, :64], m[:, 64:]]
                biases += [b[:64], b[64:]]
            wst[t] = jnp.stack(mats)
            bst[t] = jnp.stack(biases)
        return wst, bst

    x = {t: jnp.pad(x_raw[t], ((0, 0), (0, 8 - x_raw[t].shape[1])))
         for t in TYPES}
    for l in range(2):
        wst, bst = stacks(l)
        y = {t: _mm_stack(x[t], wst[t], bst[t]) for t in TYPES}
        tables = []
        for q in range(12):
            r = q // 2
            t = RELS[r][0]
            pos = SRC_OF[t].index(r)
            tables.append(y[t][2 * pos + q % 2])
        tbl = jnp.concatenate(tables, axis=0)
        s_all = _segsum_sc(tbl, idxc, zf)
        xn = {}
        for t in TYPES:
            bsum = sum(convs[l][r]["bl"] for r in DST_OF[t]).reshape(1, H)
            xn[t] = _combine(s_all, c_all, y[t], bsum,
                             [DOFF[r] for r in DST_OF[t]],
                             float(len(DST_OF[t])), NNODE[t])
        x = xn

    z = jnp.concatenate([x["agv"], x["picker"], x["location"]], axis=0)
    z = _ode(z,
             p["ode1"]["W"], p["ode1"]["b"].reshape(1, OH),
             p["ode2"]["W"], p["ode2"]["b"].reshape(1, OH),
             p["ode3"]["W"], p["ode3"]["b"].reshape(1, H))
    agv_e = z[:N_AGV]
    picker_e = z[N_AGV:N_AGV + N_PICKER]
    loc_e = z[N_AGV + N_PICKER:]
    agv_q = _head(agv_e, p["agv_h1"]["W"], p["agv_h1"]["b"].reshape(1, OH),
                  p["agv_h2"]["W"], p["agv_h2"]["b"].reshape(1, A))
    picker_q = _head(picker_e, p["picker_h1"]["W"],
                     p["picker_h1"]["b"].reshape(1, OH),
                     p["picker_h2"]["W"], p["picker_h2"]["b"].reshape(1, A))
    return (agv_q, picker_q, agv_e, picker_e, loc_e)


# speedup vs baseline: 2.8397x; 1.5112x over previous
import jax
import jax.numpy as jnp
from jax import lax
from jax.experimental import pallas as pl
from jax.experimental.pallas import tpu as pltpu
from jax.experimental.pallas import tpu_sc as plsc

H = 128
OH = 64
A = 16
N_AGV = 20000
N_PICKER = 5000
N_LOC = 20000
E = 320000
NNODE = {"agv": N_AGV, "picker": N_PICKER, "location": N_LOC}
RELS = [("agv", "location"), ("location", "agv"), ("agv", "agv"),
        ("picker", "location"), ("agv", "picker"), ("picker", "agv")]
SRC_OF = {"agv": [0, 2, 4], "location": [1], "picker": [3, 5]}
DST_OF = {"location": [0, 3], "agv": [1, 2, 5], "picker": [4]}
TYPES = ["agv", "picker", "location"]

NC, NS = 2, 16
EP = 327680
EROWS = EP // 128
ROWS_PER_TILE = EROWS // (NC * NS)
G = 4
NGROUP = ROWS_PER_TILE // G
ACC_ROWS = 20008
DUMMY = 20000
RPT = 1256
FQ = 32

DOFF = [0, 20000, 40000, 60000, 80000, 85000]
DTOT = 105000
TBASE = [0, 80000, 160000, 240000, 260000, 340000]
TTOT = 360000


def _rel_params(r):
    nd = jnp.where(r == 4, 5000, 20000)
    roff = 20000 * jnp.minimum(r, 4) + 5000 * jnp.maximum(r - 4, 0)
    return nd, roff


def _segsum_body(tblp, idxc, zf, s_out, acc, tbl_s, idxv, rows, sem_g, ss0,
                 ss1):
    c = lax.axis_index("c")
    s = lax.axis_index("s")
    tile = c * NS + s
    gid0 = tile * NGROUP

    def do_group(r, gg, b, ssem):
        pltpu.sync_copy(idxc.at[r].at[gg], idxv.at[b])
        cps = [
            pltpu.async_copy(tbl_s.at[idxv.at[b].at[j]],
                             rows.at[b].at[pl.ds(j * 128, 128)], sem_g)
            for j in range(G)
        ]
        for cp in cps:
            cp.wait()
        for j in range(G):
            pltpu.async_copy(rows.at[b].at[pl.ds(j * 128, 128)],
                             acc.at[idxv.at[b].at[G + j]], ssem, add=True)

    def drain(b, ssem):
        for j in range(G):
            pltpu.make_async_copy(
                rows.at[b].at[pl.ds(j * 128, 128)],
                acc.at[idxv.at[b].at[G + j]], ssem).wait()

    def p_body(p, carry):
        r = p // 4
        f = p % 4
        nd, roff = _rel_params(r)
        ns = jnp.where((r == 3) | (r == 5), 5000, 20000)
        tbase = (80000 * jnp.minimum(r, 3) + 20000 * jnp.maximum(r - 3, 0)
                 + 60000 * jnp.maximum(r - 4, 0) + f * ns)
        zstart = jnp.minimum(s * RPT, (nd + 8) - RPT)
        pltpu.sync_copy(zf.at[pl.ds(zstart, RPT)], acc.at[pl.ds(zstart, RPT)])
        tstart = jnp.minimum(s * RPT, ns - RPT)
        pltpu.sync_copy(tblp.at[pl.ds(tbase + tstart, RPT)],
                        tbl_s.at[pl.ds(tstart, RPT)])
        plsc.subcore_barrier()

        do_group(r, gid0 + 0, 0, ss0)
        do_group(r, gid0 + 1, 1, ss1)

        def g_body(g2, carry2):
            drain(0, ss0)
            do_group(r, gid0 + 2 * g2, 0, ss0)
            drain(1, ss1)
            do_group(r, gid0 + 2 * g2 + 1, 1, ss1)
            return carry2

        lax.fori_loop(1, NGROUP // 2, g_body, 0)
        drain(0, ss0)
        drain(1, ss1)
        plsc.subcore_barrier()
        dstart = jnp.minimum(s * RPT, nd - RPT)
        pltpu.sync_copy(acc.at[pl.ds(dstart, RPT)],
                        s_out.at[c].at[f].at[pl.ds(roff + dstart, RPT)])
        plsc.subcore_barrier()
        return carry

    lax.fori_loop(0, 24, p_body, 0)


def _counts_body(dstr, zc, ones_h, c_out, acc, idx_d, ones_v, sem):
    del sem
    c = lax.axis_index("c")
    s = lax.axis_index("s")
    row0 = (c * NS + s) * ROWS_PER_TILE
    pltpu.sync_copy(ones_h, ones_v)

    def r_body(r, carry):
        nd, roff = _rel_params(r)
        zstart = jnp.minimum(s * RPT, (nd + 8) - RPT)
        pltpu.sync_copy(zc.at[pl.ds(zstart, RPT)], acc.at[pl.ds(zstart, RPT)])
        plsc.subcore_barrier()

        def g_body(g, carry2):
            roff_rows = row0 + g * 8
            pltpu.sync_copy(dstr.at[r].at[pl.ds(roff_rows, 8)], idx_d)
            for j in range(8):
                pltpu.sync_copy(ones_v, acc.at[idx_d.at[j]], add=True)
            return carry2

        lax.fori_loop(0, ROWS_PER_TILE // 8, g_body, 0)
        plsc.subcore_barrier()
        dstart = jnp.minimum(s * RPT, nd - RPT)
        pltpu.sync_copy(acc.at[pl.ds(dstart, RPT)],
                        c_out.at[c].at[pl.ds(roff + dstart, RPT)])
        plsc.subcore_barrier()
        return carry

    lax.fori_loop(0, 6, r_body, 0)


_SC_MESH = plsc.VectorSubcoreMesh(core_axis_name="c", subcore_axis_name="s")
_SC_PARAMS = pltpu.CompilerParams(use_tc_tiling_on_sc=False)

_segsum_sc = pl.kernel(
    _segsum_body,
    out_type=jax.ShapeDtypeStruct((NC, 4, DTOT, FQ), jnp.float32),
    mesh=_SC_MESH,
    scratch_types=[
        pltpu.VMEM_SHARED((ACC_ROWS, FQ), jnp.float32),
        pltpu.VMEM_SHARED((20000, FQ), jnp.float32),
        pltpu.VMEM((2, 2 * G, 128), jnp.int32),
        pltpu.VMEM((2, G * 128, FQ), jnp.float32),
        pltpu.SemaphoreType.DMA,
        pltpu.SemaphoreType.DMA,
        pltpu.SemaphoreType.DMA,
    ],
    compiler_params=_SC_PARAMS,
)

_counts_sc = pl.kernel(
    _counts_body,
    out_type=jax.ShapeDtypeStruct((NC, DTOT, 16), jnp.float32),
    mesh=_SC_MESH,
    scratch_types=[
        pltpu.VMEM_SHARED((ACC_ROWS, 16), jnp.float32),
        pltpu.VMEM((8, 128), jnp.int32),
        pltpu.VMEM((128, 16), jnp.float32),
        pltpu.SemaphoreType.DMA,
    ],
    compiler_params=_SC_PARAMS,
)


def _mm_stack(x, wstack, bstack, bm=1000):
    n, kin = x.shape
    S = wstack.shape[0]

    def body(x_ref, w_ref, b_ref, o_ref):
        o_ref[0] = (jnp.dot(x_ref[...], w_ref[0],
                            preferred_element_type=jnp.float32) + b_ref[0])

    return pl.pallas_call(
        body,
        grid=(S, n // bm),
        in_specs=[
            pl.BlockSpec((bm, kin), lambda j, i: (i, 0)),
            pl.BlockSpec((1, kin, FQ), lambda j, i: (j, 0, 0)),
            pl.BlockSpec((1, 1, FQ), lambda j, i: (j, 0, 0)),
        ],
        out_specs=pl.BlockSpec((1, bm, FQ), lambda j, i: (j, i, 0)),
        out_shape=jax.ShapeDtypeStruct((S, n, FQ), jnp.float32),
    )(x, wstack, bstack.reshape(S, 1, FQ))


def _combine(s_all, c_all, y_self, bsum, rel_offs, kd, n, bm=1000):
    nr = len(rel_offs)
    self_blk = (y_self.shape[0] - 4) // 4

    def body(*refs):
        s_refs = refs[:nr]
        c_refs = refs[nr:2 * nr]
        sref, bref, oref = refs[2 * nr], refs[2 * nr + 1], refs[2 * nr + 2]
        tot = jnp.concatenate([sref[f] for f in range(4)], axis=-1) + bref[...]
        for s_ref, c_ref in zip(s_refs, c_refs):
            m = jnp.concatenate([s_ref[0, f] + s_ref[1, f] for f in range(4)],
                                axis=-1)
            cc = c_ref[0, :, 0] + c_ref[1, :, 0]
            tot = tot + m * (1.0 / jnp.maximum(cc, 1.0))[:, None]
        oref[...] = jnp.maximum(tot * (1.0 / kd), 0.0)

    in_specs = []
    for off in rel_offs:
        blk = off // bm
        in_specs.append(pl.BlockSpec((NC, 4, bm, FQ),
                                     lambda i, blk=blk: (0, 0, blk + i, 0)))
    for off in rel_offs:
        blk = off // bm
        in_specs.append(pl.BlockSpec((NC, bm, 16),
                                     lambda i, blk=blk: (0, blk + i, 0)))
    in_specs.append(pl.BlockSpec((4, bm, FQ), lambda i: (self_blk, i, 0)))
    in_specs.append(pl.BlockSpec((1, H), lambda i: (0, 0)))

    return pl.pallas_call(
        body,
        grid=(n // bm,),
        in_specs=in_specs,
        out_specs=pl.BlockSpec((bm, H), lambda i: (i, 0)),
        out_shape=jax.ShapeDtypeStruct((n, H), jnp.float32),
    )(*([s_all] * nr + [c_all] * nr + [y_self, bsum]))


def _ode(z, w1, b1, w2, b2, w3, b3, bm=1000):
    n = z.shape[0]

    def body(z_ref, w1r, b1r, w2r, b2r, w3r, b3r, o_ref):
        def f(h):
            h1 = jnp.tanh(jnp.dot(h, w1r[...],
                                  preferred_element_type=jnp.float32) + b1r[...])
            h2 = jnp.tanh(jnp.dot(h1, w2r[...],
                                  preferred_element_type=jnp.float32) + b2r[...])
            return jnp.dot(h2, w3r[...],
                           preferred_element_type=jnp.float32) + b3r[...]

        dt = 0.1

        def step(i, zz):
            k1 = f(zz)
            k2 = f(zz + (0.5 * dt) * k1)
            k3 = f(zz + (0.5 * dt) * k2)
            k4 = f(zz + dt * k3)
            return zz + (dt / 6.0) * (k1 + 2.0 * k2 + 2.0 * k3 + k4)

        o_ref[...] = lax.fori_loop(0, 10, step, z_ref[...])

    full = lambda shape: pl.BlockSpec(shape, lambda i: tuple(0 for _ in shape))
    return pl.pallas_call(
        body,
        grid=(n // bm,),
        in_specs=[
            pl.BlockSpec((bm, H), lambda i: (i, 0)),
            full((H, OH)), full((1, OH)), full((OH, OH)), full((1, OH)),
            full((OH, H)), full((1, H)),
        ],
        out_specs=pl.BlockSpec((bm, H), lambda i: (i, 0)),
        out_shape=jax.ShapeDtypeStruct((n, H), jnp.float32),
    )(z, w1, b1, w2, b2, w3, b3)


def _head(x, w1, b1, w2, b2, bm=1000):
    n = x.shape[0]

    def body(x_ref, w1r, b1r, w2r, b2r, o_ref):
        h = jnp.maximum(jnp.dot(x_ref[...], w1r[...],
                                preferred_element_type=jnp.float32) + b1r[...],
                        0.0)
        o_ref[...] = jnp.dot(h, w2r[...],
                             preferred_element_type=jnp.float32) + b2r[...]

    full = lambda shape: pl.BlockSpec(shape, lambda i: tuple(0 for _ in shape))
    return pl.pallas_call(
        body,
        grid=(n // bm,),
        in_specs=[
            pl.BlockSpec((bm, H), lambda i: (i, 0)),
            full((H, OH)), full((1, OH)), full((OH, A)), full((1, A)),
        ],
        out_specs=pl.BlockSpec((bm, A), lambda i: (i, 0)),
        out_shape=jax.ShapeDtypeStruct((n, A), jnp.float32),
    )(x, w1, b1, w2, b2)


def kernel(x_agv, x_picker, x_location, src0, dst0, src1, dst1, src2, dst2,
           src3, dst3, src4, dst4, src5, dst5, params):
    p = params
    convs = p["convs"]
    srcs = [src0, src1, src2, src3, src4, src5]
    dsts = [dst0, dst1, dst2, dst3, dst4, dst5]
    x_raw = {"agv": x_agv, "picker": x_picker, "location": x_location}

    pad_s = jnp.zeros((EP - E,), jnp.int32)
    pad_d = jnp.full((EP - E,), DUMMY, jnp.int32)
    src_p = [jnp.concatenate([s, pad_s]) for s in srcs]
    dst_p = [jnp.concatenate([d, pad_d]) for d in dsts]
    dstr = jnp.stack([d.reshape(EROWS, 128) for d in dst_p])
    idxc = jnp.stack([
        jnp.concatenate([
            src_p[r].reshape(EROWS // G, G, 128),
            dst_p[r].reshape(EROWS // G, G, 128),
        ], axis=1) for r in range(6)
    ])

    zf = jnp.zeros((ACC_ROWS, FQ), jnp.float32)
    zc = jnp.zeros((ACC_ROWS, 16), jnp.float32)
    ones_h = jnp.ones((128, 16), jnp.float32)

    c_all = _counts_sc(dstr, zc, ones_h)

    def stacks(l):
        wst, bst = {}, {}
        for t in TYPES:
            mats, biases = [], []
            rels_s = SRC_OF[t]
            wr_sum = sum(convs[l][r]["Wr"] for r in DST_OF[t])
            if l == 0:
                din = x_raw[t].shape[1]
                we = jnp.pad(p["emb_" + t]["W"], ((0, 8 - din), (0, 0)))
                be = p["emb_" + t]["b"]
                srcmats = [we @ convs[l][r]["Wl"] for r in rels_s]
                srcbias = [be @ convs[l][r]["Wl"] for r in rels_s]
                selfmat, selfbias = we @ wr_sum, be @ wr_sum
            else:
                srcmats = [convs[l][r]["Wl"] for r in rels_s]
                srcbias = [jnp.zeros((H,), jnp.float32) for r in rels_s]
                selfmat = wr_sum
                selfbias = jnp.zeros((H,), jnp.float32)
            for m, b in zip(srcmats + [selfmat], srcbias + [selfbias]):
                for f in range(4):
                    mats.append(m[:, FQ * f:FQ * (f + 1)])
                    biases.append(b[FQ * f:FQ * (f + 1)])
            wst[t] = jnp.stack(mats)
            bst[t] = jnp.stack(biases)
        return wst, bst

    x = {t: jnp.pad(x_raw[t], ((0, 0), (0, 8 - x_raw[t].shape[1])))
         for t in TYPES}
    for l in range(2):
        wst, bst = stacks(l)
        y = {t: _mm_stack(x[t], wst[t], bst[t]) for t in TYPES}
        tables = []
        for r in range(6):
            t = RELS[r][0]
            pos = SRC_OF[t].index(r)
            for f in range(4):
                tables.append(y[t][4 * pos + f])
        tblp = jnp.concatenate(tables, axis=0)
        s_all = _segsum_sc(tblp, idxc, zf)
        xn = {}
        for t in TYPES:
            bsum = sum(convs[l][r]["bl"] for r in DST_OF[t]).reshape(1, H)
            xn[t] = _combine(s_all, c_all, y[t], bsum,
                             [DOFF[r] for r in DST_OF[t]],
                             float(len(DST_OF[t])), NNODE[t])
        x = xn

    z = jnp.concatenate([x["agv"], x["picker"], x["location"]], axis=0)
    z = _ode(z,
             p["ode1"]["W"], p["ode1"]["b"].reshape(1, OH),
             p["ode2"]["W"], p["ode2"]["b"].reshape(1, OH),
             p["ode3"]["W"], p["ode3"]["b"].reshape(1, H))
    agv_e = z[:N_AGV]
    picker_e = z[N_AGV:N_AGV + N_PICKER]
    loc_e = z[N_AGV + N_PICKER:]
    agv_q = _head(agv_e, p["agv_h1"]["W"], p["agv_h1"]["b"].reshape(1, OH),
                  p["agv_h2"]["W"], p["agv_h2"]["b"].reshape(1, A))
    picker_q = _head(picker_e, p["picker_h1"]["W"],
                     p["picker_h1"]["b"].reshape(1, OH),
                     p["picker_h2"]["W"], p["picker_h2"]["b"].reshape(1, A))
    return (agv_q, picker_q, agv_e, picker_e, loc_e)


# speedup vs baseline: 2.9113x; 1.0252x over previous
import jax
import jax.numpy as jnp
from jax import lax
from jax.experimental import pallas as pl
from jax.experimental.pallas import tpu as pltpu
from jax.experimental.pallas import tpu_sc as plsc

H = 128
OH = 64
A = 16
N_AGV = 20000
N_PICKER = 5000
N_LOC = 20000
E = 320000
NNODE = {"agv": N_AGV, "picker": N_PICKER, "location": N_LOC}
RELS = [("agv", "location"), ("location", "agv"), ("agv", "agv"),
        ("picker", "location"), ("agv", "picker"), ("picker", "agv")]
SRC_OF = {"agv": [0, 2, 4], "location": [1], "picker": [3, 5]}
DST_OF = {"location": [0, 3], "agv": [1, 2, 5], "picker": [4]}
TYPES = ["agv", "picker", "location"]

NC, NS = 2, 16
EP = 327680
EROWS = EP // 128
ROWS_PER_TILE = EROWS // (NC * NS)
G = 4
NGROUP = ROWS_PER_TILE // G
ACC_ROWS = 20008
DUMMY = 20000
RPT = 1256
FQ = 32

DOFF = [0, 20000, 40000, 60000, 80000, 85000]
DTOT = 105000
TBASE = [0, 80000, 160000, 240000, 260000, 340000]
TTOT = 360000


def _rel_params(r):
    nd = jnp.where(r == 4, 5000, 20000)
    roff = 20000 * jnp.minimum(r, 4) + 5000 * jnp.maximum(r - 4, 0)
    return nd, roff


def _segsum_body(tblp, idxc, zf, s_out, acc, tbl_s, idxv, rows, sem_g, ss0,
                 ss1):
    c = lax.axis_index("c")
    s = lax.axis_index("s")
    tile = c * NS + s
    gid0 = tile * NGROUP

    def do_group(r, gg, b, ssem):
        pltpu.sync_copy(idxc.at[r].at[gg], idxv.at[b])
        cps = [
            pltpu.async_copy(tbl_s.at[idxv.at[b].at[j]],
                             rows.at[b].at[pl.ds(j * 128, 128)], sem_g)
            for j in range(G)
        ]
        for cp in cps:
            cp.wait()
        for j in range(G):
            pltpu.async_copy(rows.at[b].at[pl.ds(j * 128, 128)],
                             acc.at[idxv.at[b].at[G + j]], ssem, add=True)

    def drain(b, ssem):
        for j in range(G):
            pltpu.make_async_copy(
                rows.at[b].at[pl.ds(j * 128, 128)],
                acc.at[idxv.at[b].at[G + j]], ssem).wait()

    def p_body(p, carry):
        r = p // 4
        f = p % 4
        nd, roff = _rel_params(r)
        ns = jnp.where((r == 3) | (r == 5), 5000, 20000)
        tbase = (80000 * jnp.minimum(r, 3) + 20000 * jnp.maximum(r - 3, 0)
                 + 60000 * jnp.maximum(r - 4, 0) + f * ns)
        zstart = jnp.minimum(s * RPT, (nd + 8) - RPT)
        pltpu.sync_copy(zf.at[pl.ds(zstart, RPT)], acc.at[pl.ds(zstart, RPT)])
        tstart = jnp.minimum(s * RPT, ns - RPT)
        pltpu.sync_copy(tblp.at[pl.ds(tbase + tstart, RPT)],
                        tbl_s.at[pl.ds(tstart, RPT)])
        plsc.subcore_barrier()

        do_group(r, gid0 + 0, 0, ss0)
        do_group(r, gid0 + 1, 1, ss1)

        def g_body(g2, carry2):
            drain(0, ss0)
            do_group(r, gid0 + 2 * g2, 0, ss0)
            drain(1, ss1)
            do_group(r, gid0 + 2 * g2 + 1, 1, ss1)
            return carry2

        lax.fori_loop(1, NGROUP // 2, g_body, 0)
        drain(0, ss0)
        drain(1, ss1)
        plsc.subcore_barrier()
        dstart = jnp.minimum(s * RPT, nd - RPT)
        pltpu.sync_copy(acc.at[pl.ds(dstart, RPT)],
                        s_out.at[c].at[f].at[pl.ds(roff + dstart, RPT)])
        plsc.subcore_barrier()
        return carry

    lax.fori_loop(0, 24, p_body, 0)


def _counts_body(dstr, zc, ones_h, c_out, acc, idx_d, ones_v, sem):
    del sem
    c = lax.axis_index("c")
    s = lax.axis_index("s")
    row0 = (c * NS + s) * ROWS_PER_TILE
    pltpu.sync_copy(ones_h, ones_v)

    def r_body(r, carry):
        nd, roff = _rel_params(r)
        zstart = jnp.minimum(s * RPT, (nd + 8) - RPT)
        pltpu.sync_copy(zc.at[pl.ds(zstart, RPT)], acc.at[pl.ds(zstart, RPT)])
        plsc.subcore_barrier()

        def g_body(g, carry2):
            roff_rows = row0 + g * 8
            pltpu.sync_copy(dstr.at[r].at[pl.ds(roff_rows, 8)], idx_d)
            for j in range(8):
                pltpu.sync_copy(ones_v, acc.at[idx_d.at[j]], add=True)
            return carry2

        lax.fori_loop(0, ROWS_PER_TILE // 8, g_body, 0)
        plsc.subcore_barrier()
        dstart = jnp.minimum(s * RPT, nd - RPT)
        pltpu.sync_copy(acc.at[pl.ds(dstart, RPT)],
                        c_out.at[c].at[pl.ds(roff + dstart, RPT)])
        plsc.subcore_barrier()
        return carry

    lax.fori_loop(0, 6, r_body, 0)


_SC_MESH = plsc.VectorSubcoreMesh(core_axis_name="c", subcore_axis_name="s")
_SC_PARAMS = pltpu.CompilerParams(use_tc_tiling_on_sc=False)

_segsum_sc = pl.kernel(
    _segsum_body,
    out_type=jax.ShapeDtypeStruct((NC, 4, DTOT, FQ), jnp.float32),
    mesh=_SC_MESH,
    scratch_types=[
        pltpu.VMEM_SHARED((ACC_ROWS, FQ), jnp.float32),
        pltpu.VMEM_SHARED((20000, FQ), jnp.float32),
        pltpu.VMEM((2, 2 * G, 128), jnp.int32),
        pltpu.VMEM((2, G * 128, FQ), jnp.float32),
        pltpu.SemaphoreType.DMA,
        pltpu.SemaphoreType.DMA,
        pltpu.SemaphoreType.DMA,
    ],
    compiler_params=_SC_PARAMS,
)

_counts_sc = pl.kernel(
    _counts_body,
    out_type=jax.ShapeDtypeStruct((NC, DTOT, 16), jnp.float32),
    mesh=_SC_MESH,
    scratch_types=[
        pltpu.VMEM_SHARED((ACC_ROWS, 16), jnp.float32),
        pltpu.VMEM((8, 128), jnp.int32),
        pltpu.VMEM((128, 16), jnp.float32),
        pltpu.SemaphoreType.DMA,
    ],
    compiler_params=_SC_PARAMS,
)


def _mm_stack(x, wstack, bstack, bm=1000):
    n, kin = x.shape
    S = wstack.shape[0]

    def body(x_ref, w_ref, b_ref, o_ref):
        o_ref[0] = (jnp.dot(x_ref[...], w_ref[0],
                            preferred_element_type=jnp.float32) + b_ref[0])

    return pl.pallas_call(
        body,
        grid=(n // bm, S),
        in_specs=[
            pl.BlockSpec((bm, kin), lambda i, j: (i, 0)),
            pl.BlockSpec((1, kin, FQ), lambda i, j: (j, 0, 0)),
            pl.BlockSpec((1, 1, FQ), lambda i, j: (j, 0, 0)),
        ],
        out_specs=pl.BlockSpec((1, bm, FQ), lambda i, j: (j, i, 0)),
        out_shape=jax.ShapeDtypeStruct((S, n, FQ), jnp.float32),
    )(x, wstack, bstack.reshape(S, 1, FQ))


def _combine(s_all, c_all, y_self, bsum, rel_offs, kd, n, bm=1000):
    nr = len(rel_offs)
    self_blk = (y_self.shape[0] - 4) // 4

    def body(*refs):
        s_refs = refs[:nr]
        c_refs = refs[nr:2 * nr]
        sref, bref, oref = refs[2 * nr], refs[2 * nr + 1], refs[2 * nr + 2]
        tot = jnp.concatenate([sref[f] for f in range(4)], axis=-1) + bref[...]
        for s_ref, c_ref in zip(s_refs, c_refs):
            m = jnp.concatenate([s_ref[0, f] + s_ref[1, f] for f in range(4)],
                                axis=-1)
            cc = c_ref[0, :, 0] + c_ref[1, :, 0]
            tot = tot + m * (1.0 / jnp.maximum(cc, 1.0))[:, None]
        oref[...] = jnp.maximum(tot * (1.0 / kd), 0.0)

    in_specs = []
    for off in rel_offs:
        blk = off // bm
        in_specs.append(pl.BlockSpec((NC, 4, bm, FQ),
                                     lambda i, blk=blk: (0, 0, blk + i, 0)))
    for off in rel_offs:
        blk = off // bm
        in_specs.append(pl.BlockSpec((NC, bm, 16),
                                     lambda i, blk=blk: (0, blk + i, 0)))
    in_specs.append(pl.BlockSpec((4, bm, FQ), lambda i: (self_blk, i, 0)))
    in_specs.append(pl.BlockSpec((1, H), lambda i: (0, 0)))

    return pl.pallas_call(
        body,
        grid=(n // bm,),
        in_specs=in_specs,
        out_specs=pl.BlockSpec((bm, H), lambda i: (i, 0)),
        out_shape=jax.ShapeDtypeStruct((n, H), jnp.float32),
    )(*([s_all] * nr + [c_all] * nr + [y_self, bsum]))


def _ode(z, w1, b1, w2, b2, w3, b3, bm=1000):
    n = z.shape[0]

    def body(z_ref, w1r, b1r, w2r, b2r, w3r, b3r, o_ref):
        def f(h):
            h1 = jnp.tanh(jnp.dot(h, w1r[...],
                                  preferred_element_type=jnp.float32) + b1r[...])
            h2 = jnp.tanh(jnp.dot(h1, w2r[...],
                                  preferred_element_type=jnp.float32) + b2r[...])
            return jnp.dot(h2, w3r[...],
                           preferred_element_type=jnp.float32) + b3r[...]

        dt = 0.1

        def step(i, zz):
            k1 = f(zz)
            k2 = f(zz + (0.5 * dt) * k1)
            k3 = f(zz + (0.5 * dt) * k2)
            k4 = f(zz + dt * k3)
            return zz + (dt / 6.0) * (k1 + 2.0 * k2 + 2.0 * k3 + k4)

        o_ref[...] = lax.fori_loop(0, 10, step, z_ref[...])

    full = lambda shape: pl.BlockSpec(shape, lambda i: tuple(0 for _ in shape))
    return pl.pallas_call(
        body,
        grid=(n // bm,),
        in_specs=[
            pl.BlockSpec((bm, H), lambda i: (i, 0)),
            full((H, OH)), full((1, OH)), full((OH, OH)), full((1, OH)),
            full((OH, H)), full((1, H)),
        ],
        out_specs=pl.BlockSpec((bm, H), lambda i: (i, 0)),
        out_shape=jax.ShapeDtypeStruct((n, H), jnp.float32),
    )(z, w1, b1, w2, b2, w3, b3)


def _head(x, w1, b1, w2, b2, bm=1000):
    n = x.shape[0]

    def body(x_ref, w1r, b1r, w2r, b2r, o_ref):
        h = jnp.maximum(jnp.dot(x_ref[...], w1r[...],
                                preferred_element_type=jnp.float32) + b1r[...],
                        0.0)
        o_ref[...] = jnp.dot(h, w2r[...],
                             preferred_element_type=jnp.float32) + b2r[...]

    full = lambda shape: pl.BlockSpec(shape, lambda i: tuple(0 for _ in shape))
    return pl.pallas_call(
        body,
        grid=(n // bm,),
        in_specs=[
            pl.BlockSpec((bm, H), lambda i: (i, 0)),
            full((H, OH)), full((1, OH)), full((OH, A)), full((1, A)),
        ],
        out_specs=pl.BlockSpec((bm, A), lambda i: (i, 0)),
        out_shape=jax.ShapeDtypeStruct((n, A), jnp.float32),
    )(x, w1, b1, w2, b2)


def kernel(x_agv, x_picker, x_location, src0, dst0, src1, dst1, src2, dst2,
           src3, dst3, src4, dst4, src5, dst5, params):
    p = params
    convs = p["convs"]
    srcs = [src0, src1, src2, src3, src4, src5]
    dsts = [dst0, dst1, dst2, dst3, dst4, dst5]
    x_raw = {"agv": x_agv, "picker": x_picker, "location": x_location}

    pad_s = jnp.zeros((EP - E,), jnp.int32)
    pad_d = jnp.full((EP - E,), DUMMY, jnp.int32)
    src_p = [jnp.concatenate([s, pad_s]) for s in srcs]
    dst_p = [jnp.concatenate([d, pad_d]) for d in dsts]
    dstr = jnp.stack([d.reshape(EROWS, 128) for d in dst_p])
    idxc = jnp.stack([
        jnp.concatenate([
            src_p[r].reshape(EROWS // G, G, 128),
            dst_p[r].reshape(EROWS // G, G, 128),
        ], axis=1) for r in range(6)
    ])

    zf = jnp.zeros((ACC_ROWS, FQ), jnp.float32)
    zc = jnp.zeros((ACC_ROWS, 16), jnp.float32)
    ones_h = jnp.ones((128, 16), jnp.float32)

    c_all = _counts_sc(dstr, zc, ones_h)

    def stacks(l):
        wst, bst = {}, {}
        for t in TYPES:
            mats, biases = [], []
            rels_s = SRC_OF[t]
            wr_sum = sum(convs[l][r]["Wr"] for r in DST_OF[t])
            if l == 0:
                din = x_raw[t].shape[1]
                we = jnp.pad(p["emb_" + t]["W"], ((0, 8 - din), (0, 0)))
                be = p["emb_" + t]["b"]
                srcmats = [we @ convs[l][r]["Wl"] for r in rels_s]
                srcbias = [be @ convs[l][r]["Wl"] for r in rels_s]
                selfmat, selfbias = we @ wr_sum, be @ wr_sum
            else:
                srcmats = [convs[l][r]["Wl"] for r in rels_s]
                srcbias = [jnp.zeros((H,), jnp.float32) for r in rels_s]
                selfmat = wr_sum
                selfbias = jnp.zeros((H,), jnp.float32)
            for m, b in zip(srcmats + [selfmat], srcbias + [selfbias]):
                for f in range(4):
                    mats.append(m[:, FQ * f:FQ * (f + 1)])
                    biases.append(b[FQ * f:FQ * (f + 1)])
            wst[t] = jnp.stack(mats)
            bst[t] = jnp.stack(biases)
        return wst, bst

    x = {t: jnp.pad(x_raw[t], ((0, 0), (0, 8 - x_raw[t].shape[1])))
         for t in TYPES}
    for l in range(2):
        wst, bst = stacks(l)
        y = {t: _mm_stack(x[t], wst[t], bst[t]) for t in TYPES}
        tables = []
        for r in range(6):
            t = RELS[r][0]
            pos = SRC_OF[t].index(r)
            for f in range(4):
                tables.append(y[t][4 * pos + f])
        tblp = jnp.concatenate(tables, axis=0)
        s_all = _segsum_sc(tblp, idxc, zf)
        xn = {}
        for t in TYPES:
            bsum = sum(convs[l][r]["bl"] for r in DST_OF[t]).reshape(1, H)
            xn[t] = _combine(s_all, c_all, y[t], bsum,
                             [DOFF[r] for r in DST_OF[t]],
                             float(len(DST_OF[t])), NNODE[t])
        x = xn

    z = jnp.concatenate([x["agv"], x["picker"], x["location"]], axis=0)
    z = _ode(z,
             p["ode1"]["W"], p["ode1"]["b"].reshape(1, OH),
             p["ode2"]["W"], p["ode2"]["b"].reshape(1, OH),
             p["ode3"]["W"], p["ode3"]["b"].reshape(1, H))
    agv_e = z[:N_AGV]
    picker_e = z[N_AGV:N_AGV + N_PICKER]
    loc_e = z[N_AGV + N_PICKER:]
    agv_q = _head(agv_e, p["agv_h1"]["W"], p["agv_h1"]["b"].reshape(1, OH),
                  p["agv_h2"]["W"], p["agv_h2"]["b"].reshape(1, A))
    picker_q = _head(picker_e, p["picker_h1"]["W"],
                     p["picker_h1"]["b"].reshape(1, OH),
                     p["picker_h2"]["W"], p["picker_h2"]["b"].reshape(1, A))
    return (agv_q, picker_q, agv_e, picker_e, loc_e)


# speedup vs baseline: 3.2700x; 1.1232x over previous
import jax
import jax.numpy as jnp
from jax import lax
from jax.experimental import pallas as pl
from jax.experimental.pallas import tpu as pltpu
from jax.experimental.pallas import tpu_sc as plsc

H = 128
OH = 64
A = 16
N_AGV = 20000
N_PICKER = 5000
N_LOC = 20000
E = 320000
NNODE = {"agv": N_AGV, "picker": N_PICKER, "location": N_LOC}
RELS = [("agv", "location"), ("location", "agv"), ("agv", "agv"),
        ("picker", "location"), ("agv", "picker"), ("picker", "agv")]
SRC_OF = {"agv": [0, 2, 4], "location": [1], "picker": [3, 5]}
DST_OF = {"location": [0, 3], "agv": [1, 2, 5], "picker": [4]}
TYPES = ["agv", "picker", "location"]

NC, NS = 2, 16
EP = 327680
EROWS = EP // 128
ROWS_PER_TILE = EROWS // (NC * NS)
G = 4
NGROUP = ROWS_PER_TILE // G
ACC_ROWS = 20008
DUMMY = 20000
RPT = 1256
FQ = 32

DOFF = [0, 20000, 40000, 60000, 80000, 85000]
DTOT = 105000
TBASE = [0, 80000, 160000, 240000, 260000, 340000]
TTOT = 360000


def _rel_params(r):
    nd = jnp.where(r == 4, 5000, 20000)
    roff = 20000 * jnp.minimum(r, 4) + 5000 * jnp.maximum(r - 4, 0)
    return nd, roff


def _sel(q, vals):
    out = jnp.int32(vals[0])
    for i in range(1, len(vals)):
        out = jnp.where(q == i, jnp.int32(vals[i]), out)
    return out


def _make_segsum_body(rlist):
    ns_list = [NNODE[RELS[r][0]] for r in rlist]
    nd_list = [NNODE[RELS[r][1]] for r in rlist]
    tb_list, rf_list = [0], [0]
    for ns_, nd_ in zip(ns_list, nd_list):
        tb_list.append(tb_list[-1] + 4 * ns_)
        rf_list.append(rf_list[-1] + nd_)

    def body(tblp, idxc, zf, s_out, acc, tbl_s, idxv, rows, sem_g, ss0, ss1):
        c = lax.axis_index("c")
        s = lax.axis_index("s")
        tile = c * NS + s
        gid0 = tile * NGROUP

        def do_group(r, gg, b, ssem):
            pltpu.sync_copy(idxc.at[r].at[gg], idxv.at[b])
            cps = [
                pltpu.async_copy(tbl_s.at[idxv.at[b].at[j]],
                                 rows.at[b].at[pl.ds(j * 128, 128)], sem_g)
                for j in range(G)
            ]
            for cp in cps:
                cp.wait()
            for j in range(G):
                pltpu.async_copy(rows.at[b].at[pl.ds(j * 128, 128)],
                                 acc.at[idxv.at[b].at[G + j]], ssem, add=True)

        def drain(b, ssem):
            for j in range(G):
                pltpu.make_async_copy(
                    rows.at[b].at[pl.ds(j * 128, 128)],
                    acc.at[idxv.at[b].at[G + j]], ssem).wait()

        def p_body(p, carry):
            q = p // 4
            f = p % 4
            r = _sel(q, rlist)
            nd = _sel(q, nd_list)
            roff = _sel(q, rf_list)
            ns = _sel(q, ns_list)
            tbase = _sel(q, tb_list) + f * ns
            zstart = jnp.minimum(s * RPT, (nd + 8) - RPT)
            pltpu.sync_copy(zf.at[pl.ds(zstart, RPT)],
                            acc.at[pl.ds(zstart, RPT)])
            tstart = jnp.minimum(s * RPT, ns - RPT)
            pltpu.sync_copy(tblp.at[pl.ds(tbase + tstart, RPT)],
                            tbl_s.at[pl.ds(tstart, RPT)])
            plsc.subcore_barrier()

            do_group(r, gid0 + 0, 0, ss0)
            do_group(r, gid0 + 1, 1, ss1)

            def g_body(g2, carry2):
                drain(0, ss0)
                do_group(r, gid0 + 2 * g2, 0, ss0)
                drain(1, ss1)
                do_group(r, gid0 + 2 * g2 + 1, 1, ss1)
                return carry2

            lax.fori_loop(1, NGROUP // 2, g_body, 0)
            drain(0, ss0)
            drain(1, ss1)
            plsc.subcore_barrier()
            dstart = jnp.minimum(s * RPT, nd - RPT)
            pltpu.sync_copy(acc.at[pl.ds(dstart, RPT)],
                            s_out.at[c].at[f].at[pl.ds(roff + dstart, RPT)])
            plsc.subcore_barrier()
            return carry

        lax.fori_loop(0, 4 * len(rlist), p_body, 0)

    return body, rf_list[-1]


def _counts_body(dstr, zc, ones_h, c_out, acc, idx_d, ones_v, sem):
    del sem
    c = lax.axis_index("c")
    s = lax.axis_index("s")
    row0 = (c * NS + s) * ROWS_PER_TILE
    pltpu.sync_copy(ones_h, ones_v)

    def r_body(r, carry):
        nd, roff = _rel_params(r)
        zstart = jnp.minimum(s * RPT, (nd + 8) - RPT)
        pltpu.sync_copy(zc.at[pl.ds(zstart, RPT)], acc.at[pl.ds(zstart, RPT)])
        plsc.subcore_barrier()

        def g_body(g, carry2):
            roff_rows = row0 + g * 8
            pltpu.sync_copy(dstr.at[r].at[pl.ds(roff_rows, 8)], idx_d)
            for j in range(8):
                pltpu.sync_copy(ones_v, acc.at[idx_d.at[j]], add=True)
            return carry2

        lax.fori_loop(0, ROWS_PER_TILE // 8, g_body, 0)
        plsc.subcore_barrier()
        dstart = jnp.minimum(s * RPT, nd - RPT)
        pltpu.sync_copy(acc.at[pl.ds(dstart, RPT)],
                        c_out.at[c].at[pl.ds(roff + dstart, RPT)])
        plsc.subcore_barrier()
        return carry

    lax.fori_loop(0, 6, r_body, 0)


_SC_MESH = plsc.VectorSubcoreMesh(core_axis_name="c", subcore_axis_name="s")
_SC_PARAMS = pltpu.CompilerParams(use_tc_tiling_on_sc=False)

RLIST_A = [0, 3]
RLIST_B = [1, 2, 4, 5]


def _make_segsum(rlist):
    body, dtot = _make_segsum_body(rlist)
    return pl.kernel(
        body,
        out_type=jax.ShapeDtypeStruct((NC, 4, dtot, FQ), jnp.float32),
        mesh=_SC_MESH,
        scratch_types=[
            pltpu.VMEM_SHARED((ACC_ROWS, FQ), jnp.float32),
            pltpu.VMEM_SHARED((20000, FQ), jnp.float32),
            pltpu.VMEM((2, 2 * G, 128), jnp.int32),
            pltpu.VMEM((2, G * 128, FQ), jnp.float32),
            pltpu.SemaphoreType.DMA,
            pltpu.SemaphoreType.DMA,
            pltpu.SemaphoreType.DMA,
        ],
        compiler_params=_SC_PARAMS,
    )


_segsum_a = _make_segsum(RLIST_A)
_segsum_b = _make_segsum(RLIST_B)
S_OFF = {0: 0, 3: 20000, 1: 0, 2: 20000, 4: 40000, 5: 45000}
S_CALL = {0: "a", 3: "a", 1: "b", 2: "b", 4: "b", 5: "b"}

_counts_sc = pl.kernel(
    _counts_body,
    out_type=jax.ShapeDtypeStruct((NC, DTOT, 16), jnp.float32),
    mesh=_SC_MESH,
    scratch_types=[
        pltpu.VMEM_SHARED((ACC_ROWS, 16), jnp.float32),
        pltpu.VMEM((8, 128), jnp.int32),
        pltpu.VMEM((128, 16), jnp.float32),
        pltpu.SemaphoreType.DMA,
    ],
    compiler_params=_SC_PARAMS,
)


def _mm_stack(x, wstack, bstack, bm=1000):
    n, kin = x.shape
    S = wstack.shape[0]

    def body(x_ref, w_ref, b_ref, o_ref):
        o_ref[0] = (jnp.dot(x_ref[...], w_ref[0],
                            preferred_element_type=jnp.float32) + b_ref[0])

    return pl.pallas_call(
        body,
        grid=(n // bm, S),
        in_specs=[
            pl.BlockSpec((bm, kin), lambda i, j: (i, 0)),
            pl.BlockSpec((1, kin, FQ), lambda i, j: (j, 0, 0)),
            pl.BlockSpec((1, 1, FQ), lambda i, j: (j, 0, 0)),
        ],
        out_specs=pl.BlockSpec((1, bm, FQ), lambda i, j: (j, i, 0)),
        out_shape=jax.ShapeDtypeStruct((S, n, FQ), jnp.float32),
    )(x, wstack, bstack.reshape(S, 1, FQ))


def _combine(s_arrs, s_offs, c_all, c_offs, y_self, bsum, kd, n, bm=1000):
    nr = len(s_offs)
    self_blk = (y_self.shape[0] - 4) // 4

    def body(*refs):
        s_refs = refs[:nr]
        c_refs = refs[nr:2 * nr]
        sref, bref, oref = refs[2 * nr], refs[2 * nr + 1], refs[2 * nr + 2]
        tot = jnp.concatenate([sref[f] for f in range(4)], axis=-1) + bref[...]
        for s_ref, c_ref in zip(s_refs, c_refs):
            m = jnp.concatenate([s_ref[0, f] + s_ref[1, f] for f in range(4)],
                                axis=-1)
            cc = c_ref[0, :, 0] + c_ref[1, :, 0]
            tot = tot + m * (1.0 / jnp.maximum(cc, 1.0))[:, None]
        oref[...] = jnp.maximum(tot * (1.0 / kd), 0.0)

    in_specs = []
    for off in s_offs:
        blk = off // bm
        in_specs.append(pl.BlockSpec((NC, 4, bm, FQ),
                                     lambda i, blk=blk: (0, 0, blk + i, 0)))
    for off in c_offs:
        blk = off // bm
        in_specs.append(pl.BlockSpec((NC, bm, 16),
                                     lambda i, blk=blk: (0, blk + i, 0)))
    in_specs.append(pl.BlockSpec((4, bm, FQ), lambda i: (self_blk, i, 0)))
    in_specs.append(pl.BlockSpec((1, H), lambda i: (0, 0)))

    return pl.pallas_call(
        body,
        grid=(n // bm,),
        in_specs=in_specs,
        out_specs=pl.BlockSpec((bm, H), lambda i: (i, 0)),
        out_shape=jax.ShapeDtypeStruct((n, H), jnp.float32),
    )(*(s_arrs + [c_all] * nr + [y_self, bsum]))


def _ode(z, w1, b1, w2, b2, w3, b3, bm=1000):
    n = z.shape[0]

    def body(z_ref, w1r, b1r, w2r, b2r, w3r, b3r, o_ref):
        def f(h):
            h1 = jnp.tanh(jnp.dot(h, w1r[...],
                                  preferred_element_type=jnp.float32) + b1r[...])
            h2 = jnp.tanh(jnp.dot(h1, w2r[...],
                                  preferred_element_type=jnp.float32) + b2r[...])
            return jnp.dot(h2, w3r[...],
                           preferred_element_type=jnp.float32) + b3r[...]

        dt = 0.1

        def step(i, zz):
            k1 = f(zz)
            k2 = f(zz + (0.5 * dt) * k1)
            k3 = f(zz + (0.5 * dt) * k2)
            k4 = f(zz + dt * k3)
            return zz + (dt / 6.0) * (k1 + 2.0 * k2 + 2.0 * k3 + k4)

        o_ref[...] = lax.fori_loop(0, 10, step, z_ref[...])

    full = lambda shape: pl.BlockSpec(shape, lambda i: tuple(0 for _ in shape))
    return pl.pallas_call(
        body,
        grid=(n // bm,),
        in_specs=[
            pl.BlockSpec((bm, H), lambda i: (i, 0)),
            full((H, OH)), full((1, OH)), full((OH, OH)), full((1, OH)),
            full((OH, H)), full((1, H)),
        ],
        out_specs=pl.BlockSpec((bm, H), lambda i: (i, 0)),
        out_shape=jax.ShapeDtypeStruct((n, H), jnp.float32),
    )(z, w1, b1, w2, b2, w3, b3)


def _head(x, w1, b1, w2, b2, bm=1000):
    n = x.shape[0]

    def body(x_ref, w1r, b1r, w2r, b2r, o_ref):
        h = jnp.maximum(jnp.dot(x_ref[...], w1r[...],
                                preferred_element_type=jnp.float32) + b1r[...],
                        0.0)
        o_ref[...] = jnp.dot(h, w2r[...],
                             preferred_element_type=jnp.float32) + b2r[...]

    full = lambda shape: pl.BlockSpec(shape, lambda i: tuple(0 for _ in shape))
    return pl.pallas_call(
        body,
        grid=(n // bm,),
        in_specs=[
            pl.BlockSpec((bm, H), lambda i: (i, 0)),
            full((H, OH)), full((1, OH)), full((OH, A)), full((1, A)),
        ],
        out_specs=pl.BlockSpec((bm, A), lambda i: (i, 0)),
        out_shape=jax.ShapeDtypeStruct((n, A), jnp.float32),
    )(x, w1, b1, w2, b2)


def kernel(x_agv, x_picker, x_location, src0, dst0, src1, dst1, src2, dst2,
           src3, dst3, src4, dst4, src5, dst5, params):
    p = params
    convs = p["convs"]
    srcs = [src0, src1, src2, src3, src4, src5]
    dsts = [dst0, dst1, dst2, dst3, dst4, dst5]
    x_raw = {"agv": x_agv, "picker": x_picker, "location": x_location}

    pad_s = jnp.zeros((EP - E,), jnp.int32)
    pad_d = jnp.full((EP - E,), DUMMY, jnp.int32)
    src_p = [jnp.concatenate([s, pad_s]) for s in srcs]
    dst_p = [jnp.concatenate([d, pad_d]) for d in dsts]
    dstr = jnp.stack([d.reshape(EROWS, 128) for d in dst_p])
    idxc = jnp.stack([
        jnp.concatenate([
            src_p[r].reshape(EROWS // G, G, 128),
            dst_p[r].reshape(EROWS // G, G, 128),
        ], axis=1) for r in range(6)
    ])

    zf = jnp.zeros((ACC_ROWS, FQ), jnp.float32)
    zc = jnp.zeros((ACC_ROWS, 16), jnp.float32)
    ones_h = jnp.ones((128, 16), jnp.float32)

    c_all = _counts_sc(dstr, zc, ones_h)

    def stacks(l):
        wst, bst = {}, {}
        for t in TYPES:
            mats, biases = [], []
            rels_s = SRC_OF[t]
            wr_sum = sum(convs[l][r]["Wr"] for r in DST_OF[t])
            if l == 0:
                din = x_raw[t].shape[1]
                we = jnp.pad(p["emb_" + t]["W"], ((0, 8 - din), (0, 0)))
                be = p["emb_" + t]["b"]
                srcmats = [we @ convs[l][r]["Wl"] for r in rels_s]
                srcbias = [be @ convs[l][r]["Wl"] for r in rels_s]
                selfmat, selfbias = we @ wr_sum, be @ wr_sum
            else:
                srcmats = [convs[l][r]["Wl"] for r in rels_s]
                srcbias = [jnp.zeros((H,), jnp.float32) for r in rels_s]
                selfmat = wr_sum
                selfbias = jnp.zeros((H,), jnp.float32)
            for m, b in zip(srcmats + [selfmat], srcbias + [selfbias]):
                for f in range(4):
                    mats.append(m[:, FQ * f:FQ * (f + 1)])
                    biases.append(b[FQ * f:FQ * (f + 1)])
            wst[t] = jnp.stack(mats)
            bst[t] = jnp.stack(biases)
        return wst, bst

    x = {t: jnp.pad(x_raw[t], ((0, 0), (0, 8 - x_raw[t].shape[1])))
         for t in TYPES}

    def run_layer(l, x):
        wst, bst = stacks(l)
        y = {t: _mm_stack(x[t], wst[t], bst[t]) for t in TYPES}

        def tbl_for(rlist):
            tables = []
            for r in rlist:
                t = RELS[r][0]
                pos = SRC_OF[t].index(r)
                for f in range(4):
                    tables.append(y[t][4 * pos + f])
            return jnp.concatenate(tables, axis=0)

        s_of = {}
        s_of["a"] = _segsum_a(tbl_for(RLIST_A), idxc, zf)
        s_of["b"] = _segsum_b(tbl_for(RLIST_B), idxc, zf)

        def comb(t):
            rels_d = DST_OF[t]
            bsum = sum(convs[l][r]["bl"] for r in rels_d).reshape(1, H)
            return _combine([s_of[S_CALL[r]] for r in rels_d],
                            [S_OFF[r] for r in rels_d],
                            c_all, [DOFF[r] for r in rels_d],
                            y[t], bsum, float(len(rels_d)), NNODE[t])

        return {"location": comb("location"), "agv": comb("agv"),
                "picker": comb("picker")}

    x = run_layer(0, x)
    x = run_layer(1, x)

    def ode(t):
        return _ode(x[t],
                    p["ode1"]["W"], p["ode1"]["b"].reshape(1, OH),
                    p["ode2"]["W"], p["ode2"]["b"].reshape(1, OH),
                    p["ode3"]["W"], p["ode3"]["b"].reshape(1, H))

    loc_e = ode("location")
    agv_e = ode("agv")
    picker_e = ode("picker")
    agv_q = _head(agv_e, p["agv_h1"]["W"], p["agv_h1"]["b"].reshape(1, OH),
                  p["agv_h2"]["W"], p["agv_h2"]["b"].reshape(1, A))
    picker_q = _head(picker_e, p["picker_h1"]["W"],
                     p["picker_h1"]["b"].reshape(1, OH),
                     p["picker_h2"]["W"], p["picker_h2"]["b"].reshape(1, A))
    return (agv_q, picker_q, agv_e, picker_e, loc_e)


# speedup vs baseline: 3.6016x; 1.1014x over previous
import jax
import jax.numpy as jnp
from jax import lax
from jax.experimental import pallas as pl
from jax.experimental.pallas import tpu as pltpu
from jax.experimental.pallas import tpu_sc as plsc

H = 128
OH = 64
A = 16
N_AGV = 20000
N_PICKER = 5000
N_LOC = 20000
E = 320000
NNODE = {"agv": N_AGV, "picker": N_PICKER, "location": N_LOC}
RELS = [("agv", "location"), ("location", "agv"), ("agv", "agv"),
        ("picker", "location"), ("agv", "picker"), ("picker", "agv")]
SRC_OF = {"agv": [0, 2, 4], "location": [1], "picker": [3, 5]}
DST_OF = {"location": [0, 3], "agv": [1, 2, 5], "picker": [4]}
TYPES = ["agv", "picker", "location"]

NC, NS = 2, 16
EP = 327680
EROWS = EP // 128
ROWS_PER_TILE = EROWS // (NC * NS)
G = 4
NGROUP = ROWS_PER_TILE // G
ACC_ROWS = 20008
DUMMY = 20000
RPT = 1256
FQ = 32

DOFF = [0, 20000, 40000, 60000, 80000, 85000]
DTOT = 105000
TBASE = [0, 80000, 160000, 240000, 260000, 340000]
TTOT = 360000


def _rel_params(r):
    nd = jnp.where(r == 4, 5000, 20000)
    roff = 20000 * jnp.minimum(r, 4) + 5000 * jnp.maximum(r - 4, 0)
    return nd, roff


def _sel(q, vals):
    out = jnp.int32(vals[0])
    for i in range(1, len(vals)):
        out = jnp.where(q == i, jnp.int32(vals[i]), out)
    return out


def _make_segsum_body(rlist):
    ns_list = [NNODE[RELS[r][0]] for r in rlist]
    nd_list = [NNODE[RELS[r][1]] for r in rlist]
    tb_list, rf_list = [0], [0]
    for ns_, nd_ in zip(ns_list, nd_list):
        tb_list.append(tb_list[-1] + 4 * ns_)
        rf_list.append(rf_list[-1] + nd_)

    def body(tblp, idxc, zf, s_out, acc, tbl_s, idxv, rows, sem_g, ss0, ss1):
        c = lax.axis_index("c")
        s = lax.axis_index("s")
        tile = c * NS + s
        gid0 = tile * NGROUP

        def do_group(r, gg, b, ssem):
            pltpu.sync_copy(idxc.at[r].at[gg], idxv.at[b])
            cps = [
                pltpu.async_copy(tbl_s.at[idxv.at[b].at[j]],
                                 rows.at[b].at[pl.ds(j * 128, 128)], sem_g)
                for j in range(G)
            ]
            for cp in cps:
                cp.wait()
            for j in range(G):
                pltpu.async_copy(rows.at[b].at[pl.ds(j * 128, 128)],
                                 acc.at[idxv.at[b].at[G + j]], ssem, add=True)

        def drain(b, ssem):
            for j in range(G):
                pltpu.make_async_copy(
                    rows.at[b].at[pl.ds(j * 128, 128)],
                    acc.at[idxv.at[b].at[G + j]], ssem).wait()

        def p_body(p, carry):
            q = p // 4
            f = p % 4
            r = _sel(q, rlist)
            nd = _sel(q, nd_list)
            roff = _sel(q, rf_list)
            ns = _sel(q, ns_list)
            tbase = _sel(q, tb_list) + f * ns
            zstart = jnp.minimum(s * RPT, (nd + 8) - RPT)
            pltpu.sync_copy(zf.at[pl.ds(zstart, RPT)],
                            acc.at[pl.ds(zstart, RPT)])
            tstart = jnp.minimum(s * RPT, ns - RPT)
            pltpu.sync_copy(tblp.at[pl.ds(tbase + tstart, RPT)],
                            tbl_s.at[pl.ds(tstart, RPT)])
            plsc.subcore_barrier()

            do_group(r, gid0 + 0, 0, ss0)
            do_group(r, gid0 + 1, 1, ss1)

            def g_body(g2, carry2):
                drain(0, ss0)
                do_group(r, gid0 + 2 * g2, 0, ss0)
                drain(1, ss1)
                do_group(r, gid0 + 2 * g2 + 1, 1, ss1)
                return carry2

            lax.fori_loop(1, NGROUP // 2, g_body, 0)
            drain(0, ss0)
            drain(1, ss1)
            plsc.subcore_barrier()
            dstart = jnp.minimum(s * RPT, nd - RPT)
            pltpu.sync_copy(acc.at[pl.ds(dstart, RPT)],
                            s_out.at[c].at[f].at[pl.ds(roff + dstart, RPT)])
            plsc.subcore_barrier()
            return carry

        lax.fori_loop(0, 4 * len(rlist), p_body, 0)

    return body, rf_list[-1]


def _counts_body(dstr, zc, ones_h, c_out, acc, idx_d, ones_v, sem):
    del sem
    c = lax.axis_index("c")
    s = lax.axis_index("s")
    row0 = (c * NS + s) * ROWS_PER_TILE
    pltpu.sync_copy(ones_h, ones_v)

    def r_body(r, carry):
        nd, roff = _rel_params(r)
        zstart = jnp.minimum(s * RPT, (nd + 8) - RPT)
        pltpu.sync_copy(zc.at[pl.ds(zstart, RPT)], acc.at[pl.ds(zstart, RPT)])
        plsc.subcore_barrier()

        def g_body(g, carry2):
            roff_rows = row0 + g * 8
            pltpu.sync_copy(dstr.at[r].at[pl.ds(roff_rows, 8)], idx_d)
            for j in range(8):
                pltpu.sync_copy(ones_v, acc.at[idx_d.at[j]], add=True)
            return carry2

        lax.fori_loop(0, ROWS_PER_TILE // 8, g_body, 0)
        plsc.subcore_barrier()
        dstart = jnp.minimum(s * RPT, nd - RPT)
        pltpu.sync_copy(acc.at[pl.ds(dstart, RPT)],
                        c_out.at[c].at[pl.ds(roff + dstart, RPT)])
        plsc.subcore_barrier()
        return carry

    lax.fori_loop(0, 6, r_body, 0)


_SC_MESH = plsc.VectorSubcoreMesh(core_axis_name="c", subcore_axis_name="s")
_SC_PARAMS = pltpu.CompilerParams(use_tc_tiling_on_sc=False)

RLIST_A = [1, 2, 5]
RLIST_B = [0, 3, 4]


def _make_segsum(rlist):
    body, dtot = _make_segsum_body(rlist)
    return pl.kernel(
        body,
        out_type=jax.ShapeDtypeStruct((NC, 4, dtot, FQ), jnp.float32),
        mesh=_SC_MESH,
        scratch_types=[
            pltpu.VMEM_SHARED((ACC_ROWS, FQ), jnp.float32),
            pltpu.VMEM_SHARED((20000, FQ), jnp.float32),
            pltpu.VMEM((2, 2 * G, 128), jnp.int32),
            pltpu.VMEM((2, G * 128, FQ), jnp.float32),
            pltpu.SemaphoreType.DMA,
            pltpu.SemaphoreType.DMA,
            pltpu.SemaphoreType.DMA,
        ],
        compiler_params=_SC_PARAMS,
    )


_segsum_a = _make_segsum(RLIST_A)
_segsum_b = _make_segsum(RLIST_B)
S_OFF = {1: 0, 2: 20000, 5: 40000, 0: 0, 3: 20000, 4: 40000}
S_CALL = {1: "a", 2: "a", 5: "a", 0: "b", 3: "b", 4: "b"}

_counts_sc = pl.kernel(
    _counts_body,
    out_type=jax.ShapeDtypeStruct((NC, DTOT, 16), jnp.float32),
    mesh=_SC_MESH,
    scratch_types=[
        pltpu.VMEM_SHARED((ACC_ROWS, 16), jnp.float32),
        pltpu.VMEM((8, 128), jnp.int32),
        pltpu.VMEM((128, 16), jnp.float32),
        pltpu.SemaphoreType.DMA,
    ],
    compiler_params=_SC_PARAMS,
)


def _mm_stack(x, wstack, bstack, bm=1000):
    n, kin = x.shape
    S = wstack.shape[0]

    def body(x_ref, w_ref, b_ref, o_ref):
        o_ref[0] = (jnp.dot(x_ref[...], w_ref[0],
                            preferred_element_type=jnp.float32) + b_ref[0])

    return pl.pallas_call(
        body,
        grid=(n // bm, S),
        in_specs=[
            pl.BlockSpec((bm, kin), lambda i, j: (i, 0)),
            pl.BlockSpec((1, kin, FQ), lambda i, j: (j, 0, 0)),
            pl.BlockSpec((1, 1, FQ), lambda i, j: (j, 0, 0)),
        ],
        out_specs=pl.BlockSpec((1, bm, FQ), lambda i, j: (j, i, 0)),
        out_shape=jax.ShapeDtypeStruct((S, n, FQ), jnp.float32),
    )(x, wstack, bstack.reshape(S, 1, FQ))


def _combine(s_arrs, s_offs, c_all, c_offs, y_self, bsum, kd, n, bm=1000):
    nr = len(s_offs)
    self_blk = (y_self.shape[0] - 4) // 4

    def body(*refs):
        s_refs = refs[:nr]
        c_refs = refs[nr:2 * nr]
        sref, bref, oref = refs[2 * nr], refs[2 * nr + 1], refs[2 * nr + 2]
        tot = jnp.concatenate([sref[f] for f in range(4)], axis=-1) + bref[...]
        for s_ref, c_ref in zip(s_refs, c_refs):
            m = jnp.concatenate([s_ref[0, f] + s_ref[1, f] for f in range(4)],
                                axis=-1)
            cc = c_ref[0, :, 0] + c_ref[1, :, 0]
            tot = tot + m * (1.0 / jnp.maximum(cc, 1.0))[:, None]
        oref[...] = jnp.maximum(tot * (1.0 / kd), 0.0)

    in_specs = []
    for off in s_offs:
        blk = off // bm
        in_specs.append(pl.BlockSpec((NC, 4, bm, FQ),
                                     lambda i, blk=blk: (0, 0, blk + i, 0)))
    for off in c_offs:
        blk = off // bm
        in_specs.append(pl.BlockSpec((NC, bm, 16),
                                     lambda i, blk=blk: (0, blk + i, 0)))
    in_specs.append(pl.BlockSpec((4, bm, FQ), lambda i: (self_blk, i, 0)))
    in_specs.append(pl.BlockSpec((1, H), lambda i: (0, 0)))

    return pl.pallas_call(
        body,
        grid=(n // bm,),
        in_specs=in_specs,
        out_specs=pl.BlockSpec((bm, H), lambda i: (i, 0)),
        out_shape=jax.ShapeDtypeStruct((n, H), jnp.float32),
    )(*(s_arrs + [c_all] * nr + [y_self, bsum]))


def _ode(z, w1, b1, w2, b2, w3, b3, bm=1000):
    n = z.shape[0]

    def body(z_ref, w1r, b1r, w2r, b2r, w3r, b3r, o_ref):
        def f(h):
            h1 = jnp.tanh(jnp.dot(h, w1r[...],
                                  preferred_element_type=jnp.float32) + b1r[...])
            h2 = jnp.tanh(jnp.dot(h1, w2r[...],
                                  preferred_element_type=jnp.float32) + b2r[...])
            return jnp.dot(h2, w3r[...],
                           preferred_element_type=jnp.float32) + b3r[...]

        dt = 0.1

        def step(i, zz):
            k1 = f(zz)
            k2 = f(zz + (0.5 * dt) * k1)
            k3 = f(zz + (0.5 * dt) * k2)
            k4 = f(zz + dt * k3)
            return zz + (dt / 6.0) * (k1 + 2.0 * k2 + 2.0 * k3 + k4)

        o_ref[...] = lax.fori_loop(0, 10, step, z_ref[...])

    full = lambda shape: pl.BlockSpec(shape, lambda i: tuple(0 for _ in shape))
    return pl.pallas_call(
        body,
        grid=(n // bm,),
        in_specs=[
            pl.BlockSpec((bm, H), lambda i: (i, 0)),
            full((H, OH)), full((1, OH)), full((OH, OH)), full((1, OH)),
            full((OH, H)), full((1, H)),
        ],
        out_specs=pl.BlockSpec((bm, H), lambda i: (i, 0)),
        out_shape=jax.ShapeDtypeStruct((n, H), jnp.float32),
    )(z, w1, b1, w2, b2, w3, b3)


def _head(x, w1, b1, w2, b2, bm=1000):
    n = x.shape[0]

    def body(x_ref, w1r, b1r, w2r, b2r, o_ref):
        h = jnp.maximum(jnp.dot(x_ref[...], w1r[...],
                                preferred_element_type=jnp.float32) + b1r[...],
                        0.0)
        o_ref[...] = jnp.dot(h, w2r[...],
                             preferred_element_type=jnp.float32) + b2r[...]

    full = lambda shape: pl.BlockSpec(shape, lambda i: tuple(0 for _ in shape))
    return pl.pallas_call(
        body,
        grid=(n // bm,),
        in_specs=[
            pl.BlockSpec((bm, H), lambda i: (i, 0)),
            full((H, OH)), full((1, OH)), full((OH, A)), full((1, A)),
        ],
        out_specs=pl.BlockSpec((bm, A), lambda i: (i, 0)),
        out_shape=jax.ShapeDtypeStruct((n, A), jnp.float32),
    )(x, w1, b1, w2, b2)


def kernel(x_agv, x_picker, x_location, src0, dst0, src1, dst1, src2, dst2,
           src3, dst3, src4, dst4, src5, dst5, params):
    p = params
    convs = p["convs"]
    srcs = [src0, src1, src2, src3, src4, src5]
    dsts = [dst0, dst1, dst2, dst3, dst4, dst5]
    x_raw = {"agv": x_agv, "picker": x_picker, "location": x_location}

    pad_s = jnp.zeros((EP - E,), jnp.int32)
    pad_d = jnp.full((EP - E,), DUMMY, jnp.int32)
    src_p = [jnp.concatenate([s, pad_s]) for s in srcs]
    dst_p = [jnp.concatenate([d, pad_d]) for d in dsts]
    dstr = jnp.stack([d.reshape(EROWS, 128) for d in dst_p])
    idxc = jnp.stack([
        jnp.concatenate([
            src_p[r].reshape(EROWS // G, G, 128),
            dst_p[r].reshape(EROWS // G, G, 128),
        ], axis=1) for r in range(6)
    ])

    zf = jnp.zeros((ACC_ROWS, FQ), jnp.float32)
    zc = jnp.zeros((ACC_ROWS, 16), jnp.float32)
    ones_h = jnp.ones((128, 16), jnp.float32)

    c_all = _counts_sc(dstr, zc, ones_h)

    def stacks(l):
        wst, bst = {}, {}
        for t in TYPES:
            mats, biases = [], []
            rels_s = SRC_OF[t]
            wr_sum = sum(convs[l][r]["Wr"] for r in DST_OF[t])
            if l == 0:
                din = x_raw[t].shape[1]
                we = jnp.pad(p["emb_" + t]["W"], ((0, 8 - din), (0, 0)))
                be = p["emb_" + t]["b"]
                srcmats = [we @ convs[l][r]["Wl"] for r in rels_s]
                srcbias = [be @ convs[l][r]["Wl"] for r in rels_s]
                selfmat, selfbias = we @ wr_sum, be @ wr_sum
            else:
                srcmats = [convs[l][r]["Wl"] for r in rels_s]
                srcbias = [jnp.zeros((H,), jnp.float32) for r in rels_s]
                selfmat = wr_sum
                selfbias = jnp.zeros((H,), jnp.float32)
            for m, b in zip(srcmats + [selfmat], srcbias + [selfbias]):
                for f in range(4):
                    mats.append(m[:, FQ * f:FQ * (f + 1)])
                    biases.append(b[FQ * f:FQ * (f + 1)])
            wst[t] = jnp.stack(mats)
            bst[t] = jnp.stack(biases)
        return wst, bst

    x = {t: jnp.pad(x_raw[t], ((0, 0), (0, 8 - x_raw[t].shape[1])))
         for t in TYPES}

    def run_layer(l, x):
        wst, bst = stacks(l)
        y = {t: _mm_stack(x[t], wst[t], bst[t]) for t in TYPES}

        def tbl_for(rlist):
            tables = []
            for r in rlist:
                t = RELS[r][0]
                pos = SRC_OF[t].index(r)
                for f in range(4):
                    tables.append(y[t][4 * pos + f])
            return jnp.concatenate(tables, axis=0)

        s_of = {}
        s_of["a"] = _segsum_a(tbl_for(RLIST_A), idxc, zf)
        s_of["b"] = _segsum_b(tbl_for(RLIST_B), idxc, zf)

        def comb(t):
            rels_d = DST_OF[t]
            bsum = sum(convs[l][r]["bl"] for r in rels_d).reshape(1, H)
            return _combine([s_of[S_CALL[r]] for r in rels_d],
                            [S_OFF[r] for r in rels_d],
                            c_all, [DOFF[r] for r in rels_d],
                            y[t], bsum, float(len(rels_d)), NNODE[t])

        return {"agv": comb("agv"), "location": comb("location"),
                "picker": comb("picker")}

    x = run_layer(0, x)
    x = run_layer(1, x)

    def ode(t):
        return _ode(x[t],
                    p["ode1"]["W"], p["ode1"]["b"].reshape(1, OH),
                    p["ode2"]["W"], p["ode2"]["b"].reshape(1, OH),
                    p["ode3"]["W"], p["ode3"]["b"].reshape(1, H))

    agv_e = ode("agv")
    loc_e = ode("location")
    picker_e = ode("picker")
    agv_q = _head(agv_e, p["agv_h1"]["W"], p["agv_h1"]["b"].reshape(1, OH),
                  p["agv_h2"]["W"], p["agv_h2"]["b"].reshape(1, A))
    picker_q = _head(picker_e, p["picker_h1"]["W"],
                     p["picker_h1"]["b"].reshape(1, OH),
                     p["picker_h2"]["W"], p["picker_h2"]["b"].reshape(1, A))
    return (agv_q, picker_q, agv_e, picker_e, loc_e)


# speedup vs baseline: 3.8299x; 1.0634x over previous
import jax
import jax.numpy as jnp
from jax import lax
from jax.experimental import pallas as pl
from jax.experimental.pallas import tpu as pltpu
from jax.experimental.pallas import tpu_sc as plsc

H = 128
OH = 64
A = 16
N_AGV = 20000
N_PICKER = 5000
N_LOC = 20000
E = 320000
NNODE = {"agv": N_AGV, "picker": N_PICKER, "location": N_LOC}
RELS = [("agv", "location"), ("location", "agv"), ("agv", "agv"),
        ("picker", "location"), ("agv", "picker"), ("picker", "agv")]
SRC_OF = {"agv": [0, 2, 4], "location": [1], "picker": [3, 5]}
DST_OF = {"location": [0, 3], "agv": [1, 2, 5], "picker": [4]}
TYPES = ["agv", "picker", "location"]

NC, NS = 2, 16
EP = 327680
EROWS = EP // 128
ROWS_PER_TILE = EROWS // (NC * NS)
G = 4
NGROUP = ROWS_PER_TILE // G
ACC_ROWS = 20008
DUMMY = 20000
RPT = 1256
FQ = 32

DOFF = [0, 20000, 40000, 60000, 80000, 85000]
DTOT = 105000
TBASE = [0, 80000, 160000, 240000, 260000, 340000]
TTOT = 360000


def _rel_params(r):
    nd = jnp.where(r == 4, 5000, 20000)
    roff = 20000 * jnp.minimum(r, 4) + 5000 * jnp.maximum(r - 4, 0)
    return nd, roff


def _sel(q, vals):
    out = jnp.int32(vals[0])
    for i in range(1, len(vals)):
        out = jnp.where(q == i, jnp.int32(vals[i]), out)
    return out


def _make_segsum_body(rlist):
    ns_list = [NNODE[RELS[r][0]] for r in rlist]
    nd_list = [NNODE[RELS[r][1]] for r in rlist]
    tb_list, rf_list = [0], [0]
    for ns_, nd_ in zip(ns_list, nd_list):
        tb_list.append(tb_list[-1] + 4 * ns_)
        rf_list.append(rf_list[-1] + nd_)

    def body(tblp, idxc, zf, s_out, acc, tbl_s, idxv, rows, sem_g, ss0, ss1):
        c = lax.axis_index("c")
        s = lax.axis_index("s")
        tile = c * NS + s
        gid0 = tile * NGROUP

        def do_group(r, gg, b, ssem):
            pltpu.sync_copy(idxc.at[r].at[gg], idxv.at[b])
            cps = [
                pltpu.async_copy(tbl_s.at[idxv.at[b].at[j]],
                                 rows.at[b].at[pl.ds(j * 128, 128)], sem_g)
                for j in range(G)
            ]
            for cp in cps:
                cp.wait()
            for j in range(G):
                pltpu.async_copy(rows.at[b].at[pl.ds(j * 128, 128)],
                                 acc.at[idxv.at[b].at[G + j]], ssem, add=True)

        def drain(b, ssem):
            for j in range(G):
                pltpu.make_async_copy(
                    rows.at[b].at[pl.ds(j * 128, 128)],
                    acc.at[idxv.at[b].at[G + j]], ssem).wait()

        def p_body(p, carry):
            q = p // 4
            f = p % 4
            r = _sel(q, rlist)
            nd = _sel(q, nd_list)
            roff = _sel(q, rf_list)
            ns = _sel(q, ns_list)
            tbase = _sel(q, tb_list) + f * ns
            zstart = jnp.minimum(s * RPT, (nd + 8) - RPT)
            cpz = pltpu.async_copy(zf.at[pl.ds(zstart, RPT)],
                                   acc.at[pl.ds(zstart, RPT)], sem_g)
            tstart = jnp.minimum(s * RPT, ns - RPT)
            cpt = pltpu.async_copy(tblp.at[pl.ds(tbase + tstart, RPT)],
                                   tbl_s.at[pl.ds(tstart, RPT)], sem_g)
            cpz.wait()
            cpt.wait()
            plsc.subcore_barrier()

            do_group(r, gid0 + 0, 0, ss0)
            do_group(r, gid0 + 1, 1, ss1)

            def g_body(g2, carry2):
                drain(0, ss0)
                do_group(r, gid0 + 2 * g2, 0, ss0)
                drain(1, ss1)
                do_group(r, gid0 + 2 * g2 + 1, 1, ss1)
                return carry2

            lax.fori_loop(1, NGROUP // 2, g_body, 0)
            drain(0, ss0)
            drain(1, ss1)
            plsc.subcore_barrier()
            dstart = jnp.minimum(s * RPT, nd - RPT)
            pltpu.sync_copy(acc.at[pl.ds(dstart, RPT)],
                            s_out.at[c].at[f].at[pl.ds(roff + dstart, RPT)])
            plsc.subcore_barrier()
            return carry

        lax.fori_loop(0, 4 * len(rlist), p_body, 0)

    return body, rf_list[-1]


def _counts_body(dstr, zc, ones_h, c_out, acc, idx_d, ones_v, sem):
    del sem
    c = lax.axis_index("c")
    s = lax.axis_index("s")
    row0 = (c * NS + s) * ROWS_PER_TILE
    pltpu.sync_copy(ones_h, ones_v)

    def r_body(r, carry):
        nd, roff = _rel_params(r)
        zstart = jnp.minimum(s * RPT, (nd + 8) - RPT)
        pltpu.sync_copy(zc.at[pl.ds(zstart, RPT)], acc.at[pl.ds(zstart, RPT)])
        plsc.subcore_barrier()

        def g_body(g, carry2):
            roff_rows = row0 + g * 8
            pltpu.sync_copy(dstr.at[r].at[pl.ds(roff_rows, 8)], idx_d)
            for j in range(8):
                pltpu.sync_copy(ones_v, acc.at[idx_d.at[j]], add=True)
            return carry2

        lax.fori_loop(0, ROWS_PER_TILE // 8, g_body, 0)
        plsc.subcore_barrier()
        dstart = jnp.minimum(s * RPT, nd - RPT)
        pltpu.sync_copy(acc.at[pl.ds(dstart, RPT)],
                        c_out.at[c].at[pl.ds(roff + dstart, RPT)])
        plsc.subcore_barrier()
        return carry

    lax.fori_loop(0, 6, r_body, 0)


_SC_MESH = plsc.VectorSubcoreMesh(core_axis_name="c", subcore_axis_name="s")
_SC_PARAMS = pltpu.CompilerParams(use_tc_tiling_on_sc=False)

RLIST_A = [1, 2, 5]
RLIST_B = [0, 3, 4]


def _make_segsum(rlist):
    body, dtot = _make_segsum_body(rlist)
    return pl.kernel(
        body,
        out_type=jax.ShapeDtypeStruct((NC, 4, dtot, FQ), jnp.float32),
        mesh=_SC_MESH,
        scratch_types=[
            pltpu.VMEM_SHARED((ACC_ROWS, FQ), jnp.float32),
            pltpu.VMEM_SHARED((20000, FQ), jnp.float32),
            pltpu.VMEM((2, 2 * G, 128), jnp.int32),
            pltpu.VMEM((2, G * 128, FQ), jnp.float32),
            pltpu.SemaphoreType.DMA,
            pltpu.SemaphoreType.DMA,
            pltpu.SemaphoreType.DMA,
        ],
        compiler_params=_SC_PARAMS,
    )


_segsum_a = _make_segsum(RLIST_A)
_segsum_b = _make_segsum(RLIST_B)
S_OFF = {1: 0, 2: 20000, 5: 40000, 0: 0, 3: 20000, 4: 40000}
S_CALL = {1: "a", 2: "a", 5: "a", 0: "b", 3: "b", 4: "b"}

_counts_sc = pl.kernel(
    _counts_body,
    out_type=jax.ShapeDtypeStruct((NC, DTOT, 16), jnp.float32),
    mesh=_SC_MESH,
    scratch_types=[
        pltpu.VMEM_SHARED((ACC_ROWS, 16), jnp.float32),
        pltpu.VMEM((8, 128), jnp.int32),
        pltpu.VMEM((128, 16), jnp.float32),
        pltpu.SemaphoreType.DMA,
    ],
    compiler_params=_SC_PARAMS,
)


def _mm_stack(x, wstack, bstack, bm=1000):
    n, kin = x.shape
    S = wstack.shape[0]

    def body(x_ref, w_ref, b_ref, o_ref):
        o_ref[0] = (jnp.dot(x_ref[...], w_ref[0],
                            preferred_element_type=jnp.float32) + b_ref[0])

    return pl.pallas_call(
        body,
        grid=(n // bm, S),
        in_specs=[
            pl.BlockSpec((bm, kin), lambda i, j: (i, 0)),
            pl.BlockSpec((1, kin, FQ), lambda i, j: (j, 0, 0)),
            pl.BlockSpec((1, 1, FQ), lambda i, j: (j, 0, 0)),
        ],
        out_specs=pl.BlockSpec((1, bm, FQ), lambda i, j: (j, i, 0)),
        out_shape=jax.ShapeDtypeStruct((S, n, FQ), jnp.float32),
    )(x, wstack, bstack.reshape(S, 1, FQ))


def _combine(s_arrs, s_offs, c_all, c_offs, y_self, bsum, kd, n, bm=1000):
    nr = len(s_offs)
    self_blk = (y_self.shape[0] - 4) // 4

    def body(*refs):
        s_refs = refs[:nr]
        c_refs = refs[nr:2 * nr]
        sref, bref, oref = refs[2 * nr], refs[2 * nr + 1], refs[2 * nr + 2]
        tot = jnp.concatenate([sref[f] for f in range(4)], axis=-1) + bref[...]
        for s_ref, c_ref in zip(s_refs, c_refs):
            m = jnp.concatenate([s_ref[0, f] + s_ref[1, f] for f in range(4)],
                                axis=-1)
            cc = c_ref[0, :, 0] + c_ref[1, :, 0]
            tot = tot + m * (1.0 / jnp.maximum(cc, 1.0))[:, None]
        oref[...] = jnp.maximum(tot * (1.0 / kd), 0.0)

    in_specs = []
    for off in s_offs:
        blk = off // bm
        in_specs.append(pl.BlockSpec((NC, 4, bm, FQ),
                                     lambda i, blk=blk: (0, 0, blk + i, 0)))
    for off in c_offs:
        blk = off // bm
        in_specs.append(pl.BlockSpec((NC, bm, 16),
                                     lambda i, blk=blk: (0, blk + i, 0)))
    in_specs.append(pl.BlockSpec((4, bm, FQ), lambda i: (self_blk, i, 0)))
    in_specs.append(pl.BlockSpec((1, H), lambda i: (0, 0)))

    return pl.pallas_call(
        body,
        grid=(n // bm,),
        in_specs=in_specs,
        out_specs=pl.BlockSpec((bm, H), lambda i: (i, 0)),
        out_shape=jax.ShapeDtypeStruct((n, H), jnp.float32),
    )(*(s_arrs + [c_all] * nr + [y_self, bsum]))


def _ode(z, w1, b1, w2, b2, w3, b3, bm=1000):
    n = z.shape[0]

    def body(z_ref, w1r, b1r, w2r, b2r, w3r, b3r, o_ref):
        def f(h):
            h1 = jnp.tanh(jnp.dot(h, w1r[...],
                                  preferred_element_type=jnp.float32) + b1r[...])
            h2 = jnp.tanh(jnp.dot(h1, w2r[...],
                                  preferred_element_type=jnp.float32) + b2r[...])
            return jnp.dot(h2, w3r[...],
                           preferred_element_type=jnp.float32) + b3r[...]

        dt = 0.1

        def step(i, zz):
            k1 = f(zz)
            k2 = f(zz + (0.5 * dt) * k1)
            k3 = f(zz + (0.5 * dt) * k2)
            k4 = f(zz + dt * k3)
            return zz + (dt / 6.0) * (k1 + 2.0 * k2 + 2.0 * k3 + k4)

        o_ref[...] = lax.fori_loop(0, 10, step, z_ref[...])

    full = lambda shape: pl.BlockSpec(shape, lambda i: tuple(0 for _ in shape))
    return pl.pallas_call(
        body,
        grid=(n // bm,),
        in_specs=[
            pl.BlockSpec((bm, H), lambda i: (i, 0)),
            full((H, OH)), full((1, OH)), full((OH, OH)), full((1, OH)),
            full((OH, H)), full((1, H)),
        ],
        out_specs=pl.BlockSpec((bm, H), lambda i: (i, 0)),
        out_shape=jax.ShapeDtypeStruct((n, H), jnp.float32),
    )(z, w1, b1, w2, b2, w3, b3)


def _head(x, w1, b1, w2, b2, bm=1000):
    n = x.shape[0]

    def body(x_ref, w1r, b1r, w2r, b2r, o_ref):
        h = jnp.maximum(jnp.dot(x_ref[...], w1r[...],
                                preferred_element_type=jnp.float32) + b1r[...],
                        0.0)
        o_ref[...] = jnp.dot(h, w2r[...],
                             preferred_element_type=jnp.float32) + b2r[...]

    full = lambda shape: pl.BlockSpec(shape, lambda i: tuple(0 for _ in shape))
    return pl.pallas_call(
        body,
        grid=(n // bm,),
        in_specs=[
            pl.BlockSpec((bm, H), lambda i: (i, 0)),
            full((H, OH)), full((1, OH)), full((OH, A)), full((1, A)),
        ],
        out_specs=pl.BlockSpec((bm, A), lambda i: (i, 0)),
        out_shape=jax.ShapeDtypeStruct((n, A), jnp.float32),
    )(x, w1, b1, w2, b2)


def kernel(x_agv, x_picker, x_location, src0, dst0, src1, dst1, src2, dst2,
           src3, dst3, src4, dst4, src5, dst5, params):
    p = params
    convs = p["convs"]
    srcs = [src0, src1, src2, src3, src4, src5]
    dsts = [dst0, dst1, dst2, dst3, dst4, dst5]
    x_raw = {"agv": x_agv, "picker": x_picker, "location": x_location}

    pad_s = jnp.zeros((EP - E,), jnp.int32)
    pad_d = jnp.full((EP - E,), DUMMY, jnp.int32)
    src_p = [jnp.concatenate([s, pad_s]) for s in srcs]
    dst_p = [jnp.concatenate([d, pad_d]) for d in dsts]
    dstr = jnp.stack([d.reshape(EROWS, 128) for d in dst_p])
    idxc = jnp.stack([
        jnp.concatenate([
            src_p[r].reshape(EROWS // G, G, 128),
            dst_p[r].reshape(EROWS // G, G, 128),
        ], axis=1) for r in range(6)
    ])

    zf = jnp.zeros((ACC_ROWS, FQ), jnp.float32)
    zc = jnp.zeros((ACC_ROWS, 16), jnp.float32)
    ones_h = jnp.ones((128, 16), jnp.float32)

    c_all = _counts_sc(dstr, zc, ones_h)

    def stacks(l):
        wst, bst = {}, {}
        for t in TYPES:
            mats, biases = [], []
            rels_s = SRC_OF[t]
            wr_sum = sum(convs[l][r]["Wr"] for r in DST_OF[t])
            if l == 0:
                din = x_raw[t].shape[1]
                we = jnp.pad(p["emb_" + t]["W"], ((0, 8 - din), (0, 0)))
                be = p["emb_" + t]["b"]
                srcmats = [we @ convs[l][r]["Wl"] for r in rels_s]
                srcbias = [be @ convs[l][r]["Wl"] for r in rels_s]
                selfmat, selfbias = we @ wr_sum, be @ wr_sum
            else:
                srcmats = [convs[l][r]["Wl"] for r in rels_s]
                srcbias = [jnp.zeros((H,), jnp.float32) for r in rels_s]
                selfmat = wr_sum
                selfbias = jnp.zeros((H,), jnp.float32)
            for m, b in zip(srcmats + [selfmat], srcbias + [selfbias]):
                for f in range(4):
                    mats.append(m[:, FQ * f:FQ * (f + 1)])
                    biases.append(b[FQ * f:FQ * (f + 1)])
            wst[t] = jnp.stack(mats)
            bst[t] = jnp.stack(biases)
        return wst, bst

    x = {t: jnp.pad(x_raw[t], ((0, 0), (0, 8 - x_raw[t].shape[1])))
         for t in TYPES}

    def run_layer(l, x):
        wst, bst = stacks(l)
        y = {t: _mm_stack(x[t], wst[t], bst[t]) for t in TYPES}

        def tbl_for(rlist):
            tables = []
            for r in rlist:
                t = RELS[r][0]
                pos = SRC_OF[t].index(r)
                for f in range(4):
                    tables.append(y[t][4 * pos + f])
            return jnp.concatenate(tables, axis=0)

        s_of = {}
        s_of["a"] = _segsum_a(tbl_for(RLIST_A), idxc, zf)
        s_of["b"] = _segsum_b(tbl_for(RLIST_B), idxc, zf)

        def comb(t):
            rels_d = DST_OF[t]
            bsum = sum(convs[l][r]["bl"] for r in rels_d).reshape(1, H)
            return _combine([s_of[S_CALL[r]] for r in rels_d],
                            [S_OFF[r] for r in rels_d],
                            c_all, [DOFF[r] for r in rels_d],
                            y[t], bsum, float(len(rels_d)), NNODE[t])

        return {"agv": comb("agv"), "location": comb("location"),
                "picker": comb("picker")}

    x = run_layer(0, x)
    x = run_layer(1, x)

    def ode(t):
        return _ode(x[t],
                    p["ode1"]["W"], p["ode1"]["b"].reshape(1, OH),
                    p["ode2"]["W"], p["ode2"]["b"].reshape(1, OH),
                    p["ode3"]["W"], p["ode3"]["b"].reshape(1, H),
                    bm=2000 if NNODE[t] % 2000 == 0 else 1000)

    agv_e = ode("agv")
    loc_e = ode("location")
    picker_e = ode("picker")
    agv_q = _head(agv_e, p["agv_h1"]["W"], p["agv_h1"]["b"].reshape(1, OH),
                  p["agv_h2"]["W"], p["agv_h2"]["b"].reshape(1, A))
    picker_q = _head(picker_e, p["picker_h1"]["W"],
                     p["picker_h1"]["b"].reshape(1, OH),
                     p["picker_h2"]["W"], p["picker_h2"]["b"].reshape(1, A))
    return (agv_q, picker_q, agv_e, picker_e, loc_e)
